# Initial kernel scaffold; baseline (speedup 1.0000x reference)
import jax
import jax.numpy as jnp
from jax.experimental import pallas as pl


def kernel(feat, edge_index, W_head, W_tail, W_ent, attn, ln1_g, ln1_b, ln2_g, ln2_b, W_ff1, b_ff1, W_ff2, b_ff2):
    raise NotImplementedError("write your pallas kernel here")


# speedup vs baseline: 15.3061x; 15.3061x over previous
import functools

import jax
import jax.numpy as jnp
from jax import lax
from jax.experimental import pallas as pl
from jax.experimental.pallas import tpu as pltpu
from jax.experimental.pallas import tpu_sc as plsc

N = 10000
E = 320000
D = 128
H = 8
DH = 16
HOP = 5
ALPHA = 0.1
TOPK = 5
SLOPE = 0.2

NT = 32
RNG = 313
ESH = E // NT
CAPB = 512
CAPM = 16384
CAPS = 2048
NEG = -3.0e38
BIG = 3.0e38

_SC_PARAMS = pltpu.CompilerParams(needs_layout_passes=False,
                                  use_tc_tiling_on_sc=False)
_MESH = plsc.VectorSubcoreMesh(core_axis_name="c", subcore_axis_name="s")
_GDN = jax.lax.GatherDimensionNumbers((), (0,), (0,))
_IN_BOUNDS = jax.lax.GatherScatterMode.PROMISE_IN_BOUNDS


def _i16():
  return lax.iota(jnp.int32, 16)


def _gath16(v, idx):
  return jax.lax.gather(v, idx[:, None], _GDN, (1,), mode=_IN_BOUNDS)


def _seg_rmw(tab, dl, val, op):
  iota = _i16()
  sk, sv = plsc.sort_key_val(dl, iota)
  pv = _gath16(val, sv)
  for s in (1, 2, 4, 8):
    src_lane = jnp.maximum(iota - s, 0)
    same = jnp.logical_and(_gath16(sk, src_lane) == sk, iota >= s)
    shifted = _gath16(pv, src_lane)
    if op == "max":
      pv = jnp.where(same, jnp.maximum(pv, shifted), pv)
    else:
      pv = pv + jnp.where(same, shifted, jnp.zeros_like(pv))
  nxt = _gath16(sk, jnp.minimum(iota + 1, 15))
  lastm = jnp.logical_or(iota == 15, sk != nxt)
  cur = plsc.load_gather(tab, [sk])
  nv = jnp.maximum(cur, pv) if op == "max" else cur + pv
  plsc.store_scatter(tab, [sk], nv, mask=lastm)


def _pre_body(feat_ref, wh_ref, wt_ref, we_ref, g_ref, b_ref,
              fh_ref, ft_ref, f0_ref):
  x = feat_ref[...]
  mu = jnp.mean(x, axis=-1, keepdims=True)
  var = jnp.mean(jnp.square(x - mu), axis=-1, keepdims=True)
  xn = (x - mu) * jax.lax.rsqrt(var + 1e-5) * g_ref[...] + b_ref[...]
  fh_ref[...] = jnp.dot(xn, wh_ref[...], preferred_element_type=jnp.float32)
  ft_ref[...] = jnp.dot(xn, wt_ref[...], preferred_element_type=jnp.float32)
  fe = jnp.dot(xn, we_ref[...], preferred_element_type=jnp.float32)
  for h in range(H):
    f0_ref[h] = fe[:, h * DH:(h + 1) * DH]


def _k_pre(feat, wh, wt, we, g, b):
  bn = 400
  return pl.pallas_call(
      _pre_body,
      grid=(N // bn,),
      in_specs=[pl.BlockSpec((bn, D), lambda i: (i, 0)),
                pl.BlockSpec((D, D), lambda i: (0, 0)),
                pl.BlockSpec((D, D), lambda i: (0, 0)),
                pl.BlockSpec((D, D), lambda i: (0, 0)),
                pl.BlockSpec((D,), lambda i: (0,)),
                pl.BlockSpec((D,), lambda i: (0,))],
      out_specs=[pl.BlockSpec((bn, D), lambda i: (i, 0)),
                 pl.BlockSpec((bn, D), lambda i: (i, 0)),
                 pl.BlockSpec((H, bn, DH), lambda i: (0, i, 0))],
      out_shape=[jax.ShapeDtypeStruct((N, D), jnp.float32),
                 jax.ShapeDtypeStruct((N, D), jnp.float32),
                 jax.ShapeDtypeStruct((H, N, DH), jnp.float32)],
  )(feat, wh, wt, we, g, b)


@functools.partial(
    pl.kernel, mesh=_MESH, compiler_params=_SC_PARAMS,
    out_type=(jax.ShapeDtypeStruct((NT, NT, CAPB), jnp.int32),
              jax.ShapeDtypeStruct((NT, NT), jnp.int32)),
    scratch_types=[pltpu.VMEM((ESH,), jnp.int32),
                   pltpu.VMEM((NT, CAPB), jnp.int32),
                   pltpu.VMEM((48,), jnp.int32)],
)
def _k_bin(dst_hbm, bins_o, cnt_o, shard_v, bins_v, cnt_v):
  tid = lax.axis_index("c") * 16 + lax.axis_index("s")
  pltpu.sync_copy(dst_hbm.at[pl.ds(tid * ESH, ESH)], shard_v)
  for k in range(2):
    cnt_v[pl.ds(16 * k, 16)] = jnp.zeros((16,), jnp.int32)
  iota = _i16()

  def body(j, _):
    d = shard_v[pl.ds(j * 16, 16)]
    r = d // RNG
    dl = d - r * RNG
    eid = tid * ESH + j * 16 + iota
    word = eid * 512 + dl
    sk, sv = plsc.sort_key_val(r, word)
    bnd = jnp.logical_or(iota == 0, _gath16(sk, jnp.maximum(iota - 1, 0)) != sk)
    first = plsc.cummax(jnp.where(bnd, iota, -1))
    rank = iota - first
    base = plsc.load_gather(cnt_v, [sk])
    pos = jnp.minimum(base + rank, CAPB - 1)
    plsc.store_scatter(bins_v, [sk, pos], sv)
    lastm = jnp.logical_or(iota == 15, _gath16(sk, jnp.minimum(iota + 1, 15)) != sk)
    plsc.store_scatter(cnt_v, [sk], jnp.minimum(base + rank + 1, CAPB), mask=lastm)
    return 0

  lax.fori_loop(0, ESH // 16, body, 0)
  pltpu.sync_copy(bins_v, bins_o.at[tid])
  pltpu.sync_copy(cnt_v.at[pl.ds(0, NT)], cnt_o.at[tid])


@functools.partial(
    pl.kernel, mesh=_MESH, compiler_params=_SC_PARAMS,
    out_type=(jax.ShapeDtypeStruct((E, D), jnp.float32),
              jax.ShapeDtypeStruct((E, D), jnp.float32)),
    scratch_types=[pltpu.VMEM((ESH,), jnp.int32),
                   pltpu.VMEM((ESH,), jnp.int32),
                   pltpu.VMEM((128, D), jnp.float32),
                   pltpu.VMEM((128, D), jnp.float32)],
)
def _k_gath(fh_hbm, ft_hbm, src_hbm, dst_hbm, fhs_o, fts_o,
            src_v, dst_v, hbuf, tbuf):
  tid = lax.axis_index("c") * 16 + lax.axis_index("s")
  base = tid * ESH
  pltpu.sync_copy(src_hbm.at[pl.ds(base, ESH)], src_v)
  pltpu.sync_copy(dst_hbm.at[pl.ds(base, ESH)], dst_v)

  def step(off, nb):
    pltpu.sync_copy(fh_hbm.at[src_v.at[pl.ds(off, nb)]], hbuf.at[pl.ds(0, nb)])
    pltpu.sync_copy(hbuf.at[pl.ds(0, nb)], fhs_o.at[pl.ds(base + off, nb)])
    pltpu.sync_copy(ft_hbm.at[dst_v.at[pl.ds(off, nb)]], tbuf.at[pl.ds(0, nb)])
    pltpu.sync_copy(tbuf.at[pl.ds(0, nb)], fts_o.at[pl.ds(base + off, nb)])

  def body(bi, _):
    step(bi * 128, 128)
    return 0

  lax.fori_loop(0, 78, body, 0)
  step(78 * 128, 16)


def _escore_body(fhs_ref, fts_ref, attn_ref, e_ref):
  s = fhs_ref[...] + fts_ref[...]
  l = jnp.maximum(s, SLOPE * s)
  a = attn_ref[...]
  be = l.shape[0]
  for h in range(H):
    acc = jnp.zeros((be,), jnp.float32)
    for t in range(DH):
      acc = acc + l[:, h * DH + t] * a[h, t]
    e_ref[h, :] = acc


def _k_escore(fhs, fts, attn2):
  be = 2560
  return pl.pallas_call(
      _escore_body,
      grid=(E // be,),
      in_specs=[pl.BlockSpec((be, D), lambda i: (i, 0)),
                pl.BlockSpec((be, D), lambda i: (i, 0)),
                pl.BlockSpec((H, DH), lambda i: (0, 0))],
      out_specs=pl.BlockSpec((H, be), lambda i: (0, i)),
      out_shape=jax.ShapeDtypeStruct((H, E), jnp.float32),
  )(fhs, fts, attn2)


@functools.partial(
    pl.kernel, mesh=_MESH, compiler_params=_SC_PARAMS,
    out_type=(jax.ShapeDtypeStruct((H, NT, CAPS), jnp.int32),
              jax.ShapeDtypeStruct((H, NT, CAPS), jnp.int32),
              jax.ShapeDtypeStruct((H, NT, CAPS), jnp.float32),
              jax.ShapeDtypeStruct((NT, H), jnp.int32)),
    scratch_types=[pltpu.VMEM((1040,), jnp.int32),
                   pltpu.VMEM((CAPB,), jnp.int32),
                   pltpu.VMEM((CAPM + 16,), jnp.int32),
                   pltpu.VMEM((CAPM + 16,), jnp.int32),
                   pltpu.VMEM((CAPM + 16,), jnp.int32),
                   pltpu.VMEM((CAPM + 16,), jnp.int32),
                   pltpu.VMEM((CAPM + 16,), jnp.float32),
                   pltpu.VMEM((6, 320), jnp.float32),
                   pltpu.VMEM((320,), jnp.float32),
                   pltpu.VMEM((320,), jnp.float32),
                   pltpu.VMEM((320,), jnp.float32),
                   pltpu.VMEM((CAPS + 16,), jnp.int32),
                   pltpu.VMEM((CAPS + 16,), jnp.int32),
                   pltpu.VMEM((CAPS + 16,), jnp.float32),
                   pltpu.VMEM((16,), jnp.int32)],
)
def _k_seg(e_hbm, src_hbm, bins_hbm, cnt_hbm,
           ssrc_o, sdl_o, sw_o, scnt_o,
           cnt_v, bin_v, eid_v, dl_v, src_v, idx_v, ecol_v,
           tabs_v, deg_v, ctab_v, den_v, stsrc_v, stdl_v, stw_v, c8_v):
  rid = lax.axis_index("c") * 16 + lax.axis_index("s")
  iota = _i16()
  zf16 = jnp.zeros((16,), jnp.float32)
  neg16 = jnp.full((16,), NEG, jnp.float32)

  for t in range(NT):
    pltpu.sync_copy(cnt_hbm.at[t], cnt_v.at[pl.ds(t * NT, NT)])

  def zbody(j, _):
    eid_v[pl.ds(j * 16, 16)] = jnp.zeros((16,), jnp.int32)
    dl_v[pl.ds(j * 16, 16)] = jnp.zeros((16,), jnp.int32)
    return 0
  lax.fori_loop(0, (CAPM + 16) // 16, zbody, 0)

  def compact_t(t, m):
    nt = cnt_v[pl.ds(t * 32 + rid, 16)][0]
    pltpu.sync_copy(bins_hbm.at[t, rid], bin_v)

    def cbody(k, m):
      w = bin_v[pl.ds(k * 16, 16)]
      valid = k * 16 + iota < nt
      eid = jax.lax.shift_right_logical(w, 9)
      dl = jax.lax.bitwise_and(w, 511)
      plsc.store_compressed(eid_v.at[pl.ds(m, 16)], eid, mask=valid)
      plsc.store_compressed(dl_v.at[pl.ds(m, 16)], dl, mask=valid)
      return m + plsc.all_reduce_population_count(valid)[0]

    return lax.fori_loop(0, (nt + 15) // 16, cbody, m)

  m_tot = 0
  for t in range(NT):
    m_tot = compact_t(t, m_tot)
  m_tot = jnp.minimum(m_tot, CAPM)
  nch = (m_tot + 15) // 16

  for k in range(20):
    deg_v[pl.ds(k * 16, 16)] = zf16

  def degbody(j, _):
    dl = dl_v[pl.ds(j * 16, 16)]
    valid = j * 16 + iota < m_tot
    _seg_rmw(deg_v, jnp.where(valid, dl, 0),
             jnp.where(valid, 1.0, 0.0), "add")
    return 0
  lax.fori_loop(0, nch, degbody, 0)

  def logbody(k, _):
    dg = jnp.maximum(deg_v[pl.ds(k * 16, 16)], 1.0)
    bits = plsc.bitcast(dg, jnp.int32)
    ex = jax.lax.shift_right_logical(bits, 23) - 127
    mant = plsc.bitcast(jax.lax.bitwise_or(
        jax.lax.bitwise_and(bits, 0x007FFFFF), 0x3F800000), jnp.float32) - 1.0
    y = (ex.astype(jnp.float32) + mant) * 0.6931472
    for _ in range(3):
      y = y + dg * jnp.exp(-y) - 1.0
    ctab_v[pl.ds(k * 16, 16)] = y * (1.0 / DH)
    return 0
  lax.fori_loop(0, 20, logbody, 0)

  def srcb(bi, _):
    pltpu.sync_copy(src_hbm.at[eid_v.at[pl.ds(bi * 128, 128)]],
                    src_v.at[pl.ds(bi * 128, 128)])
    return 0
  lax.fori_loop(0, (m_tot + 127) // 128, srcb, 0)

  c8_v[pl.ds(0, 16)] = jnp.zeros((16,), jnp.int32)

  def head_body(h, _):
    def ib(j, _):
      idx_v[pl.ds(j * 16, 16)] = eid_v[pl.ds(j * 16, 16)] + h * E
      return 0
    lax.fori_loop(0, nch, ib, 0)

    def eb(bi, _):
      pltpu.sync_copy(e_hbm.at[idx_v.at[pl.ds(bi * 128, 128)]],
                      ecol_v.at[pl.ds(bi * 128, 128)])
      return 0
    lax.fori_loop(0, (m_tot + 127) // 128, eb, 0)
    ecol_v[pl.ds(m_tot, 16)] = neg16

    for r in range(TOPK):
      def tinit(k, _, r=r):
        tabs_v[r, pl.ds(k * 16, 16)] = neg16
        return 0
      lax.fori_loop(0, 20, tinit, 0)

      def rbody(j, _, r=r):
        dl = dl_v[pl.ds(j * 16, 16)]
        b = ecol_v[pl.ds(j * 16, 16)]
        if r == 0:
          val = b
        else:
          prev = plsc.load_gather(tabs_v.at[r - 1], [dl])
          val = jnp.where(b < prev, b, NEG)
        _seg_rmw(tabs_v.at[r], dl, val, "max")
        return 0
      lax.fori_loop(0, nch, rbody, 0)

    def dinit(k, _):
      den_v[pl.ds(k * 16, 16)] = zf16
      return 0
    lax.fori_loop(0, 20, dinit, 0)

    def dbody(j, _):
      dl = dl_v[pl.ds(j * 16, 16)]
      b = ecol_v[pl.ds(j * 16, 16)]
      thr = plsc.load_gather(tabs_v.at[TOPK - 1], [dl])
      mx = plsc.load_gather(tabs_v.at[0], [dl])
      cc = plsc.load_gather(ctab_v, [dl])
      sel = jnp.logical_and(b >= thr, b > -1.0e38)
      v = jnp.where(sel, jnp.exp(cc * (b - mx)), 0.0)
      _seg_rmw(den_v, dl, v, "add")
      return 0
    lax.fori_loop(0, nch, dbody, 0)

    def ebody(j, cnt):
      dl = dl_v[pl.ds(j * 16, 16)]
      b = ecol_v[pl.ds(j * 16, 16)]
      thr = plsc.load_gather(tabs_v.at[TOPK - 1], [dl])
      mx = plsc.load_gather(tabs_v.at[0], [dl])
      cc = plsc.load_gather(ctab_v, [dl])
      dn = plsc.load_gather(den_v, [dl])
      sel = jnp.logical_and(b >= thr, b > -1.0e38)
      w = jnp.exp(cc * (b - mx)) / jnp.maximum(dn, 1e-38)
      sv = src_v[pl.ds(j * 16, 16)]
      cnt = jnp.minimum(cnt, CAPS)
      plsc.store_compressed(stsrc_v.at[pl.ds(cnt, 16)], sv, mask=sel)
      plsc.store_compressed(stdl_v.at[pl.ds(cnt, 16)], dl, mask=sel)
      plsc.store_compressed(stw_v.at[pl.ds(cnt, 16)], w, mask=sel)
      return cnt + plsc.all_reduce_population_count(sel)[0]
    cnt = lax.fori_loop(0, nch, ebody, 0)
    cnt = jnp.minimum(cnt, CAPS)

    pltpu.sync_copy(stsrc_v.at[pl.ds(0, CAPS)], ssrc_o.at[h, rid])
    pltpu.sync_copy(stdl_v.at[pl.ds(0, CAPS)], sdl_o.at[h, rid])
    pltpu.sync_copy(stw_v.at[pl.ds(0, CAPS)], sw_o.at[h, rid])
    c8_v[pl.ds(0, 16)] = jnp.where(iota == h, cnt, c8_v[pl.ds(0, 16)])
    return 0

  lax.fori_loop(0, H, head_body, 0)
  pltpu.sync_copy(c8_v.at[pl.ds(0, H)], scnt_o.at[rid])


def _hop_body(f_hbm, f0_hbm, ssrc_hbm, sdl_hbm, sw_hbm, scnt_hbm, out_o,
              agg_v, idx_v, gbuf, sbuf, srcb_v, dlb_v, wb_v, c8_v,
              ab_v, fb_v, last):
  rid = lax.axis_index("c") * 16 + lax.axis_index("s")
  lo = rid * RNG
  iota = _i16()

  def zb(j, _):
    agg_v[pl.ds(j * 16, 16)] = jnp.zeros((16,), jnp.float32)
    return 0
  lax.fori_loop(0, (H * RNG * DH) // 16, zb, 0)

  pltpu.sync_copy(scnt_hbm.at[rid], c8_v.at[pl.ds(0, H)])
  call = c8_v[...]

  for h in range(H):
    nsel = call[h]
    pltpu.sync_copy(ssrc_hbm.at[h, rid], srcb_v)
    pltpu.sync_copy(sdl_hbm.at[h, rid], dlb_v)
    pltpu.sync_copy(sw_hbm.at[h, rid], wb_v)

    def bbody(bi, _, h=h):
      boff = bi * 128

      def isub(sub, _):
        o = boff + sub * 16
        valid = o + iota < nsel
        sv = jnp.where(valid, srcb_v[pl.ds(o, 16)], 0)
        idx_v[pl.ds(sub * 16, 16)] = sv + h * N
        return 0
      lax.fori_loop(0, 8, isub, 0)
      pltpu.sync_copy(f_hbm.at[idx_v], gbuf)

      def ssub(sub, _):
        o = boff + sub * 16
        valid = o + iota < nsel
        wv = jnp.where(valid, wb_v[pl.ds(o, 16)], 0.0)
        dv = jnp.where(valid, dlb_v[pl.ds(o, 16)], 0)
        base16 = dv * DH + h * (RNG * DH)
        for j in range(16):
          row = gbuf[sub * 16 + j, :] * wv[j]
          b = base16[j]
          agg_v[pl.ds(b, 16)] = agg_v[pl.ds(b, 16)] + row
        return 0
      lax.fori_loop(0, 8, ssub, 0)
      return 0

    lax.fori_loop(0, (nsel + 127) // 128, bbody, 0)

  for h in range(H):
    for (cb, cn) in ((0, 64), (64, 64), (128, 64), (192, 64), (256, 57)):
      pltpu.sync_copy(f0_hbm.at[pl.ds(h * N + lo + cb, cn)],
                      fb_v.at[pl.ds(0, cn)])

      def blend(j, _, h=h, cb=cb):
        a = agg_v[pl.ds(h * (RNG * DH) + (cb + j) * DH, 16)]
        f0r = fb_v[j, :]
        ab_v[j, :] = (1.0 - ALPHA) * a + ALPHA * f0r
        return 0
      lax.fori_loop(0, cn, blend, 0)
      if last:
        pltpu.sync_copy(ab_v.at[pl.ds(0, cn)],
                        out_o.at[pl.ds(lo + cb, cn), pl.ds(h * DH, DH)])
      else:
        pltpu.sync_copy(ab_v.at[pl.ds(0, cn)],
                        out_o.at[pl.ds(h * N + lo + cb, cn)])


def _make_hop(last):
  out_ty = (jax.ShapeDtypeStruct((N, D), jnp.float32) if last
            else jax.ShapeDtypeStruct((H * N, DH), jnp.float32))
  return functools.partial(
      pl.kernel, mesh=_MESH, compiler_params=_SC_PARAMS,
      out_type=out_ty,
      scratch_types=[pltpu.VMEM((H * RNG * DH,), jnp.float32),
                     pltpu.VMEM((128,), jnp.int32),
                     pltpu.VMEM((128, DH), jnp.float32),
                     pltpu.VMEM((128, DH), jnp.float32),
                     pltpu.VMEM((CAPS,), jnp.int32),
                     pltpu.VMEM((CAPS,), jnp.int32),
                     pltpu.VMEM((CAPS,), jnp.float32),
                     pltpu.VMEM((16,), jnp.int32),
                     pltpu.VMEM((64, DH), jnp.float32),
                     pltpu.VMEM((64, DH), jnp.float32)],
  )(functools.partial(_hop_body, last=last))


_k_hop_mid = _make_hop(False)
_k_hop_last = _make_hop(True)


def _ffn_body(f_ref, feat_ref, g_ref, b_ref, w1_ref, b1_ref, w2_ref, b2_ref,
              out_ref):
  rst = f_ref[...] + feat_ref[...]
  mu = jnp.mean(rst, axis=-1, keepdims=True)
  var = jnp.mean(jnp.square(rst - mu), axis=-1, keepdims=True)
  y = (rst - mu) * jax.lax.rsqrt(var + 1e-5) * g_ref[...] + b_ref[...]
  hdn = jnp.maximum(
      jnp.dot(y, w1_ref[...], preferred_element_type=jnp.float32)
      + b1_ref[...], 0.0)
  out_ref[...] = (jnp.dot(hdn, w2_ref[...], preferred_element_type=jnp.float32)
                  + b2_ref[...] + rst)


def _k_ffn(f2d, feat, g, b, w1, b1, w2, b2):
  bn = 400
  return pl.pallas_call(
      _ffn_body,
      grid=(N // bn,),
      in_specs=[pl.BlockSpec((bn, D), lambda i: (i, 0)),
                pl.BlockSpec((bn, D), lambda i: (i, 0)),
                pl.BlockSpec((D,), lambda i: (0,)),
                pl.BlockSpec((D,), lambda i: (0,)),
                pl.BlockSpec((D, 4 * D), lambda i: (0, 0)),
                pl.BlockSpec((4 * D,), lambda i: (0,)),
                pl.BlockSpec((4 * D, D), lambda i: (0, 0)),
                pl.BlockSpec((D,), lambda i: (0,))],
      out_specs=pl.BlockSpec((bn, D), lambda i: (i, 0)),
      out_shape=jax.ShapeDtypeStruct((N, D), jnp.float32),
  )(f2d, feat, g, b, w1, b1, w2, b2)


def kernel(feat, edge_index, W_head, W_tail, W_ent, attn,
           ln1_g, ln1_b, ln2_g, ln2_b, W_ff1, b_ff1, W_ff2, b_ff2):
  src = edge_index[0].astype(jnp.int32)
  dst = edge_index[1].astype(jnp.int32)
  attn2 = attn.reshape(H, DH)

  fh, ft, f0 = _k_pre(feat, W_head, W_tail, W_ent, ln1_g, ln1_b)
  f0_flat = f0.reshape(H * N, DH)

  bins, bcnt = _k_bin(dst)
  fhs, fts = _k_gath(fh, ft, src, dst)
  e = _k_escore(fhs, fts, attn2).reshape(H * E)

  ssrc, sdl, sw, scnt = _k_seg(e, src, bins, bcnt)

  f = f0_flat
  for _ in range(HOP - 1):
    f = _k_hop_mid(f, f0_flat, ssrc, sdl, sw, scnt)
  f2d = _k_hop_last(f, f0_flat, ssrc, sdl, sw, scnt)

  return _k_ffn(f2d, feat, ln2_g, ln2_b, W_ff1, b_ff1, W_ff2, b_ff2)


# speedup vs baseline: 57.8336x; 3.7785x over previous
import functools

import jax
import jax.numpy as jnp
from jax import lax
from jax.experimental import pallas as pl
from jax.experimental.pallas import tpu as pltpu
from jax.experimental.pallas import tpu_sc as plsc

N = 10000
E = 320000
D = 128
H = 8
DH = 16
HOP = 5
ALPHA = 0.1
TOPK = 5
SLOPE = 0.2

NT = 32
RNG = 313
ESH = E // NT
CAPB = 512
CAPM = 16384
CAPS = 2048
NEG = -3.0e38
BIG = 3.0e38

_SC_PARAMS = pltpu.CompilerParams(needs_layout_passes=False,
                                  use_tc_tiling_on_sc=False)
_MESH = plsc.VectorSubcoreMesh(core_axis_name="c", subcore_axis_name="s")
_GDN = jax.lax.GatherDimensionNumbers((), (0,), (0,))
_IN_BOUNDS = jax.lax.GatherScatterMode.PROMISE_IN_BOUNDS


def _i16():
  return lax.iota(jnp.int32, 16)


def _gath16(v, idx):
  return jax.lax.gather(v, idx[:, None], _GDN, (1,), mode=_IN_BOUNDS)


def _seg_rmw(tab, dl, val, op):
  iota = _i16()
  sk, sv = plsc.sort_key_val(dl, iota)
  pv = _gath16(val, sv)
  for s in (1, 2, 4, 8):
    src_lane = jnp.maximum(iota - s, 0)
    same = jnp.logical_and(_gath16(sk, src_lane) == sk, iota >= s)
    shifted = _gath16(pv, src_lane)
    if op == "max":
      pv = jnp.where(same, jnp.maximum(pv, shifted), pv)
    else:
      pv = pv + jnp.where(same, shifted, jnp.zeros_like(pv))
  nxt = _gath16(sk, jnp.minimum(iota + 1, 15))
  lastm = jnp.logical_or(iota == 15, sk != nxt)
  cur = plsc.load_gather(tab, [sk])
  nv = jnp.maximum(cur, pv) if op == "max" else cur + pv
  plsc.store_scatter(tab, [sk], nv, mask=lastm)


def _pre_body(feat_ref, wh_ref, wt_ref, we_ref, g_ref, b_ref,
              fh_ref, ft_ref, f0_ref):
  x = feat_ref[...]
  mu = jnp.mean(x, axis=-1, keepdims=True)
  var = jnp.mean(jnp.square(x - mu), axis=-1, keepdims=True)
  xn = (x - mu) * jax.lax.rsqrt(var + 1e-5) * g_ref[...] + b_ref[...]
  fh_ref[...] = jnp.dot(xn, wh_ref[...], preferred_element_type=jnp.float32)
  ft_ref[...] = jnp.dot(xn, wt_ref[...], preferred_element_type=jnp.float32)
  fe = jnp.dot(xn, we_ref[...], preferred_element_type=jnp.float32)
  for h in range(H):
    f0_ref[h] = fe[:, h * DH:(h + 1) * DH]


def _k_pre(feat, wh, wt, we, g, b):
  bn = 400
  return pl.pallas_call(
      _pre_body,
      grid=(N // bn,),
      in_specs=[pl.BlockSpec((bn, D), lambda i: (i, 0)),
                pl.BlockSpec((D, D), lambda i: (0, 0)),
                pl.BlockSpec((D, D), lambda i: (0, 0)),
                pl.BlockSpec((D, D), lambda i: (0, 0)),
                pl.BlockSpec((D,), lambda i: (0,)),
                pl.BlockSpec((D,), lambda i: (0,))],
      out_specs=[pl.BlockSpec((bn, D), lambda i: (i, 0)),
                 pl.BlockSpec((bn, D), lambda i: (i, 0)),
                 pl.BlockSpec((H, bn, DH), lambda i: (0, i, 0))],
      out_shape=[jax.ShapeDtypeStruct((N, D), jnp.float32),
                 jax.ShapeDtypeStruct((N, D), jnp.float32),
                 jax.ShapeDtypeStruct((H, N, DH), jnp.float32)],
  )(feat, wh, wt, we, g, b)


@functools.partial(
    pl.kernel, mesh=_MESH, compiler_params=_SC_PARAMS,
    out_type=(jax.ShapeDtypeStruct((NT, NT, CAPB), jnp.int32),
              jax.ShapeDtypeStruct((NT, NT), jnp.int32)),
    scratch_types=[pltpu.VMEM((ESH,), jnp.int32),
                   pltpu.VMEM((NT, CAPB), jnp.int32),
                   pltpu.VMEM((48,), jnp.int32)],
)
def _k_bin(dst_hbm, bins_o, cnt_o, shard_v, bins_v, cnt_v):
  tid = lax.axis_index("c") * 16 + lax.axis_index("s")
  pltpu.sync_copy(dst_hbm.at[pl.ds(tid * ESH, ESH)], shard_v)
  for k in range(2):
    cnt_v[pl.ds(16 * k, 16)] = jnp.zeros((16,), jnp.int32)
  iota = _i16()

  def body(j, _):
    d = shard_v[pl.ds(j * 16, 16)]
    r = d // RNG
    dl = d - r * RNG
    eid = tid * ESH + j * 16 + iota
    word = eid * 512 + dl
    sk, sv = plsc.sort_key_val(r, word)
    bnd = jnp.logical_or(iota == 0, _gath16(sk, jnp.maximum(iota - 1, 0)) != sk)
    first = plsc.cummax(jnp.where(bnd, iota, -1))
    rank = iota - first
    base = plsc.load_gather(cnt_v, [sk])
    pos = jnp.minimum(base + rank, CAPB - 1)
    plsc.store_scatter(bins_v, [sk, pos], sv)
    lastm = jnp.logical_or(iota == 15, _gath16(sk, jnp.minimum(iota + 1, 15)) != sk)
    plsc.store_scatter(cnt_v, [sk], jnp.minimum(base + rank + 1, CAPB), mask=lastm)
    return 0

  lax.fori_loop(0, ESH // 16, body, 0)
  pltpu.sync_copy(bins_v, bins_o.at[tid])
  pltpu.sync_copy(cnt_v.at[pl.ds(0, NT)], cnt_o.at[tid])


@functools.partial(
    pl.kernel, mesh=_MESH, compiler_params=_SC_PARAMS,
    out_type=(jax.ShapeDtypeStruct((E, D), jnp.float32),
              jax.ShapeDtypeStruct((E, D), jnp.float32)),
    scratch_types=[pltpu.VMEM((ESH,), jnp.int32),
                   pltpu.VMEM((ESH,), jnp.int32),
                   pltpu.VMEM((128, D), jnp.float32),
                   pltpu.VMEM((128, D), jnp.float32)],
)
def _k_gath(fh_hbm, ft_hbm, src_hbm, dst_hbm, fhs_o, fts_o,
            src_v, dst_v, hbuf, tbuf):
  tid = lax.axis_index("c") * 16 + lax.axis_index("s")
  base = tid * ESH
  pltpu.sync_copy(src_hbm.at[pl.ds(base, ESH)], src_v)
  pltpu.sync_copy(dst_hbm.at[pl.ds(base, ESH)], dst_v)

  def step(off, nb):
    pltpu.sync_copy(fh_hbm.at[src_v.at[pl.ds(off, nb)]], hbuf.at[pl.ds(0, nb)])
    pltpu.sync_copy(hbuf.at[pl.ds(0, nb)], fhs_o.at[pl.ds(base + off, nb)])
    pltpu.sync_copy(ft_hbm.at[dst_v.at[pl.ds(off, nb)]], tbuf.at[pl.ds(0, nb)])
    pltpu.sync_copy(tbuf.at[pl.ds(0, nb)], fts_o.at[pl.ds(base + off, nb)])

  def body(bi, _):
    step(bi * 128, 128)
    return 0

  lax.fori_loop(0, 78, body, 0)
  step(78 * 128, 16)


def _escore_body(fhs_ref, fts_ref, amat_ref, e_ref):
  s = fhs_ref[...] + fts_ref[...]
  l = jnp.maximum(s, SLOPE * s)
  res = jnp.dot(l, amat_ref[...], preferred_element_type=jnp.float32)
  e_ref[...] = res.T


def _k_escore(fhs, fts, amat):
  be = 2560
  return pl.pallas_call(
      _escore_body,
      grid=(E // be,),
      in_specs=[pl.BlockSpec((be, D), lambda i: (i, 0)),
                pl.BlockSpec((be, D), lambda i: (i, 0)),
                pl.BlockSpec((D, H), lambda i: (0, 0))],
      out_specs=pl.BlockSpec((H, be), lambda i: (0, i)),
      out_shape=jax.ShapeDtypeStruct((H, E), jnp.float32),
  )(fhs, fts, amat)


@functools.partial(
    pl.kernel, mesh=_MESH, compiler_params=_SC_PARAMS,
    out_type=(jax.ShapeDtypeStruct((H, NT, CAPS), jnp.int32),
              jax.ShapeDtypeStruct((H, NT, CAPS), jnp.int32),
              jax.ShapeDtypeStruct((H, NT, CAPS), jnp.float32),
              jax.ShapeDtypeStruct((NT, H), jnp.int32)),
    scratch_types=[pltpu.VMEM((1040,), jnp.int32),
                   pltpu.VMEM((CAPB,), jnp.int32),
                   pltpu.VMEM((CAPM + 16,), jnp.int32),
                   pltpu.VMEM((CAPM + 16,), jnp.int32),
                   pltpu.VMEM((CAPM + 16,), jnp.int32),
                   pltpu.VMEM((CAPM + 16,), jnp.int32),
                   pltpu.VMEM((CAPM + 16,), jnp.float32),
                   pltpu.VMEM((6, 320), jnp.float32),
                   pltpu.VMEM((320,), jnp.float32),
                   pltpu.VMEM((320,), jnp.float32),
                   pltpu.VMEM((320,), jnp.float32),
                   pltpu.VMEM((CAPS + 16,), jnp.int32),
                   pltpu.VMEM((CAPS + 16,), jnp.int32),
                   pltpu.VMEM((CAPS + 16,), jnp.float32),
                   pltpu.VMEM((16,), jnp.int32)],
)
def _k_seg(e_hbm, src_hbm, bins_hbm, cnt_hbm,
           ssrc_o, sdl_o, sw_o, scnt_o,
           cnt_v, bin_v, eid_v, dl_v, src_v, idx_v, ecol_v,
           tabs_v, deg_v, ctab_v, den_v, stsrc_v, stdl_v, stw_v, c8_v):
  rid = lax.axis_index("c") * 16 + lax.axis_index("s")
  iota = _i16()
  zf16 = jnp.zeros((16,), jnp.float32)
  neg16 = jnp.full((16,), NEG, jnp.float32)

  for t in range(NT):
    pltpu.sync_copy(cnt_hbm.at[t], cnt_v.at[pl.ds(t * NT, NT)])

  def zbody(j, _):
    eid_v[pl.ds(j * 16, 16)] = jnp.zeros((16,), jnp.int32)
    dl_v[pl.ds(j * 16, 16)] = jnp.zeros((16,), jnp.int32)
    return 0
  lax.fori_loop(0, (CAPM + 16) // 16, zbody, 0)

  def compact_t(t, m):
    nt = cnt_v[pl.ds(t * 32 + rid, 16)][0]
    pltpu.sync_copy(bins_hbm.at[t, rid], bin_v)

    def cbody(k, m):
      w = bin_v[pl.ds(k * 16, 16)]
      valid = k * 16 + iota < nt
      eid = jax.lax.shift_right_logical(w, 9)
      dl = jax.lax.bitwise_and(w, 511)
      plsc.store_compressed(eid_v.at[pl.ds(m, 16)], eid, mask=valid)
      plsc.store_compressed(dl_v.at[pl.ds(m, 16)], dl, mask=valid)
      return m + plsc.all_reduce_population_count(valid)[0]

    return lax.fori_loop(0, (nt + 15) // 16, cbody, m)

  m_tot = 0
  for t in range(NT):
    m_tot = compact_t(t, m_tot)
  m_tot = jnp.minimum(m_tot, CAPM)
  nch = (m_tot + 15) // 16

  for k in range(20):
    deg_v[pl.ds(k * 16, 16)] = zf16

  def degbody(j, _):
    dl = dl_v[pl.ds(j * 16, 16)]
    valid = j * 16 + iota < m_tot
    _seg_rmw(deg_v, jnp.where(valid, dl, 0),
             jnp.where(valid, 1.0, 0.0), "add")
    return 0
  lax.fori_loop(0, nch, degbody, 0)

  def logbody(k, _):
    dg = jnp.maximum(deg_v[pl.ds(k * 16, 16)], 1.0)
    bits = plsc.bitcast(dg, jnp.int32)
    ex = jax.lax.shift_right_logical(bits, 23) - 127
    mant = plsc.bitcast(jax.lax.bitwise_or(
        jax.lax.bitwise_and(bits, 0x007FFFFF), 0x3F800000), jnp.float32) - 1.0
    y = (ex.astype(jnp.float32) + mant) * 0.6931472
    for _ in range(3):
      y = y + dg * jnp.exp(-y) - 1.0
    ctab_v[pl.ds(k * 16, 16)] = y * (1.0 / DH)
    return 0
  lax.fori_loop(0, 20, logbody, 0)

  def srcb(bi, _):
    pltpu.sync_copy(src_hbm.at[eid_v.at[pl.ds(bi * 128, 128)]],
                    src_v.at[pl.ds(bi * 128, 128)])
    return 0
  lax.fori_loop(0, (m_tot + 127) // 128, srcb, 0)

  c8_v[pl.ds(0, 16)] = jnp.zeros((16,), jnp.int32)

  def head_body(h, _):
    def ib(j, _):
      idx_v[pl.ds(j * 16, 16)] = eid_v[pl.ds(j * 16, 16)] + h * E
      return 0
    lax.fori_loop(0, nch, ib, 0)

    def eb(bi, _):
      pltpu.sync_copy(e_hbm.at[idx_v.at[pl.ds(bi * 128, 128)]],
                      ecol_v.at[pl.ds(bi * 128, 128)])
      return 0
    lax.fori_loop(0, (m_tot + 127) // 128, eb, 0)
    ecol_v[pl.ds(m_tot, 16)] = neg16

    for r in range(TOPK):
      def tinit(k, _, r=r):
        tabs_v[r, pl.ds(k * 16, 16)] = neg16
        return 0
      lax.fori_loop(0, 20, tinit, 0)

      def rbody(j, _, r=r):
        dl = dl_v[pl.ds(j * 16, 16)]
        b = ecol_v[pl.ds(j * 16, 16)]
        if r == 0:
          val = b
        else:
          prev = plsc.load_gather(tabs_v.at[r - 1], [dl])
          val = jnp.where(b < prev, b, NEG)
        _seg_rmw(tabs_v.at[r], dl, val, "max")
        return 0
      lax.fori_loop(0, nch, rbody, 0)

    def dinit(k, _):
      den_v[pl.ds(k * 16, 16)] = zf16
      return 0
    lax.fori_loop(0, 20, dinit, 0)

    def dbody(j, _):
      dl = dl_v[pl.ds(j * 16, 16)]
      b = ecol_v[pl.ds(j * 16, 16)]
      thr = plsc.load_gather(tabs_v.at[TOPK - 1], [dl])
      mx = plsc.load_gather(tabs_v.at[0], [dl])
      cc = plsc.load_gather(ctab_v, [dl])
      sel = jnp.logical_and(b >= thr, b > -1.0e38)
      v = jnp.where(sel, jnp.exp(cc * (b - mx)), 0.0)
      _seg_rmw(den_v, dl, v, "add")
      return 0
    lax.fori_loop(0, nch, dbody, 0)

    def ebody(j, cnt):
      dl = dl_v[pl.ds(j * 16, 16)]
      b = ecol_v[pl.ds(j * 16, 16)]
      thr = plsc.load_gather(tabs_v.at[TOPK - 1], [dl])
      mx = plsc.load_gather(tabs_v.at[0], [dl])
      cc = plsc.load_gather(ctab_v, [dl])
      dn = plsc.load_gather(den_v, [dl])
      sel = jnp.logical_and(b >= thr, b > -1.0e38)
      w = jnp.exp(cc * (b - mx)) / jnp.maximum(dn, 1e-38)
      sv = src_v[pl.ds(j * 16, 16)]
      cnt = jnp.minimum(cnt, CAPS)
      plsc.store_compressed(stsrc_v.at[pl.ds(cnt, 16)], sv, mask=sel)
      plsc.store_compressed(stdl_v.at[pl.ds(cnt, 16)], dl, mask=sel)
      plsc.store_compressed(stw_v.at[pl.ds(cnt, 16)], w, mask=sel)
      return cnt + plsc.all_reduce_population_count(sel)[0]
    cnt = lax.fori_loop(0, nch, ebody, 0)
    cnt = jnp.minimum(cnt, CAPS)

    pltpu.sync_copy(stsrc_v.at[pl.ds(0, CAPS)], ssrc_o.at[h, rid])
    pltpu.sync_copy(stdl_v.at[pl.ds(0, CAPS)], sdl_o.at[h, rid])
    pltpu.sync_copy(stw_v.at[pl.ds(0, CAPS)], sw_o.at[h, rid])
    c8_v[pl.ds(0, 16)] = jnp.where(iota == h, cnt, c8_v[pl.ds(0, 16)])
    return 0

  lax.fori_loop(0, H, head_body, 0)
  pltpu.sync_copy(c8_v.at[pl.ds(0, H)], scnt_o.at[rid])


def _hop_body(f_hbm, f0_hbm, ssrc_hbm, sdl_hbm, sw_hbm, scnt_hbm, out_o,
              agg_v, idx_v, gbuf, sbuf, srcb_v, dlb_v, wb_v, c8_v,
              ab_v, fb_v, last):
  rid = lax.axis_index("c") * 16 + lax.axis_index("s")
  lo = rid * RNG
  iota = _i16()

  def zb(j, _):
    agg_v[pl.ds(j * 16, 16)] = jnp.zeros((16,), jnp.float32)
    return 0
  lax.fori_loop(0, (H * RNG * DH) // 16, zb, 0)

  pltpu.sync_copy(scnt_hbm.at[rid], c8_v.at[pl.ds(0, H)])
  call = c8_v[...]

  for h in range(H):
    nsel = call[h]
    pltpu.sync_copy(ssrc_hbm.at[h, rid], srcb_v)
    pltpu.sync_copy(sdl_hbm.at[h, rid], dlb_v)
    pltpu.sync_copy(sw_hbm.at[h, rid], wb_v)

    def bbody(bi, _, h=h):
      boff = bi * 128

      def isub(sub, _):
        o = boff + sub * 16
        valid = o + iota < nsel
        sv = jnp.where(valid, srcb_v[pl.ds(o, 16)], 0)
        idx_v[pl.ds(sub * 16, 16)] = sv + h * N
        return 0
      lax.fori_loop(0, 8, isub, 0)
      pltpu.sync_copy(f_hbm.at[idx_v], gbuf)

      def ssub(sub, _):
        o = boff + sub * 16
        valid = o + iota < nsel
        wv = jnp.where(valid, wb_v[pl.ds(o, 16)], 0.0)
        dv = jnp.where(valid, dlb_v[pl.ds(o, 16)], 0)
        base16 = dv * DH + h * (RNG * DH)
        for j in range(16):
          row = gbuf[sub * 16 + j, :] * wv[j]
          b = base16[j]
          agg_v[pl.ds(b, 16)] = agg_v[pl.ds(b, 16)] + row
        return 0
      lax.fori_loop(0, 8, ssub, 0)
      return 0

    lax.fori_loop(0, (nsel + 127) // 128, bbody, 0)

  for h in range(H):
    for (cb, cn) in ((0, 64), (64, 64), (128, 64), (192, 64), (256, 57)):
      pltpu.sync_copy(f0_hbm.at[pl.ds(h * N + lo + cb, cn)],
                      fb_v.at[pl.ds(0, cn)])

      def blend(j, _, h=h, cb=cb):
        a = agg_v[pl.ds(h * (RNG * DH) + (cb + j) * DH, 16)]
        f0r = fb_v[j, :]
        ab_v[j, :] = (1.0 - ALPHA) * a + ALPHA * f0r
        return 0
      lax.fori_loop(0, cn, blend, 0)
      if last:
        pltpu.sync_copy(ab_v.at[pl.ds(0, cn)],
                        out_o.at[pl.ds(lo + cb, cn), pl.ds(h * DH, DH)])
      else:
        pltpu.sync_copy(ab_v.at[pl.ds(0, cn)],
                        out_o.at[pl.ds(h * N + lo + cb, cn)])


def _make_hop(last):
  out_ty = (jax.ShapeDtypeStruct((N, D), jnp.float32) if last
            else jax.ShapeDtypeStruct((H * N, DH), jnp.float32))
  return functools.partial(
      pl.kernel, mesh=_MESH, compiler_params=_SC_PARAMS,
      out_type=out_ty,
      scratch_types=[pltpu.VMEM((H * RNG * DH,), jnp.float32),
                     pltpu.VMEM((128,), jnp.int32),
                     pltpu.VMEM((128, DH), jnp.float32),
                     pltpu.VMEM((128, DH), jnp.float32),
                     pltpu.VMEM((CAPS,), jnp.int32),
                     pltpu.VMEM((CAPS,), jnp.int32),
                     pltpu.VMEM((CAPS,), jnp.float32),
                     pltpu.VMEM((16,), jnp.int32),
                     pltpu.VMEM((64, DH), jnp.float32),
                     pltpu.VMEM((64, DH), jnp.float32)],
  )(functools.partial(_hop_body, last=last))


_k_hop_mid = _make_hop(False)
_k_hop_last = _make_hop(True)


def _ffn_body(f_ref, feat_ref, g_ref, b_ref, w1_ref, b1_ref, w2_ref, b2_ref,
              out_ref):
  rst = f_ref[...] + feat_ref[...]
  mu = jnp.mean(rst, axis=-1, keepdims=True)
  var = jnp.mean(jnp.square(rst - mu), axis=-1, keepdims=True)
  y = (rst - mu) * jax.lax.rsqrt(var + 1e-5) * g_ref[...] + b_ref[...]
  hdn = jnp.maximum(
      jnp.dot(y, w1_ref[...], preferred_element_type=jnp.float32)
      + b1_ref[...], 0.0)
  out_ref[...] = (jnp.dot(hdn, w2_ref[...], preferred_element_type=jnp.float32)
                  + b2_ref[...] + rst)


def _k_ffn(f2d, feat, g, b, w1, b1, w2, b2):
  bn = 400
  return pl.pallas_call(
      _ffn_body,
      grid=(N // bn,),
      in_specs=[pl.BlockSpec((bn, D), lambda i: (i, 0)),
                pl.BlockSpec((bn, D), lambda i: (i, 0)),
                pl.BlockSpec((D,), lambda i: (0,)),
                pl.BlockSpec((D,), lambda i: (0,)),
                pl.BlockSpec((D, 4 * D), lambda i: (0, 0)),
                pl.BlockSpec((4 * D,), lambda i: (0,)),
                pl.BlockSpec((4 * D, D), lambda i: (0, 0)),
                pl.BlockSpec((D,), lambda i: (0,))],
      out_specs=pl.BlockSpec((bn, D), lambda i: (i, 0)),
      out_shape=jax.ShapeDtypeStruct((N, D), jnp.float32),
  )(f2d, feat, g, b, w1, b1, w2, b2)


def kernel(feat, edge_index, W_head, W_tail, W_ent, attn,
           ln1_g, ln1_b, ln2_g, ln2_b, W_ff1, b_ff1, W_ff2, b_ff2):
  src = edge_index[0].astype(jnp.int32)
  dst = edge_index[1].astype(jnp.int32)
  attn2 = attn.reshape(H, DH)
  amat = (jnp.eye(H, dtype=jnp.float32)[:, None, :]
          * attn2[:, :, None]).reshape(D, H)

  fh, ft, f0 = _k_pre(feat, W_head, W_tail, W_ent, ln1_g, ln1_b)
  f0_flat = f0.reshape(H * N, DH)

  bins, bcnt = _k_bin(dst)
  fhs, fts = _k_gath(fh, ft, src, dst)
  e = _k_escore(fhs, fts, amat).reshape(H * E)

  ssrc, sdl, sw, scnt = _k_seg(e, src, bins, bcnt)

  f = f0_flat
  for _ in range(HOP - 1):
    f = _k_hop_mid(f, f0_flat, ssrc, sdl, sw, scnt)
  f2d = _k_hop_last(f, f0_flat, ssrc, sdl, sw, scnt)

  return _k_ffn(f2d, feat, ln2_g, ln2_b, W_ff1, b_ff1, W_ff2, b_ff2)


# speedup vs baseline: 75.4634x; 1.3048x over previous
import functools

import jax
import jax.numpy as jnp
from jax import lax
from jax.experimental import pallas as pl
from jax.experimental.pallas import tpu as pltpu
from jax.experimental.pallas import tpu_sc as plsc

N = 10000
E = 320000
D = 128
H = 8
DH = 16
HOP = 5
ALPHA = 0.1
TOPK = 5
SLOPE = 0.2

NT = 32
RNG = 313
ESH = E // NT
CAPB = 512
CAPM = 16384
CAPS = 2048
NEG = -3.0e38
BIG = 3.0e38

_SC_PARAMS = pltpu.CompilerParams(needs_layout_passes=False,
                                  use_tc_tiling_on_sc=False)
_MESH = plsc.VectorSubcoreMesh(core_axis_name="c", subcore_axis_name="s")
_GDN = jax.lax.GatherDimensionNumbers((), (0,), (0,))
_IN_BOUNDS = jax.lax.GatherScatterMode.PROMISE_IN_BOUNDS


def _i16():
  return lax.iota(jnp.int32, 16)


def _gath16(v, idx):
  return jax.lax.gather(v, idx[:, None], _GDN, (1,), mode=_IN_BOUNDS)


def _seg_rmw(tab, dl, val, op):
  iota = _i16()
  sk, sv = plsc.sort_key_val(dl, iota)
  pv = _gath16(val, sv)
  for s in (1, 2, 4, 8):
    src_lane = jnp.maximum(iota - s, 0)
    same = jnp.logical_and(_gath16(sk, src_lane) == sk, iota >= s)
    shifted = _gath16(pv, src_lane)
    if op == "max":
      pv = jnp.where(same, jnp.maximum(pv, shifted), pv)
    else:
      pv = pv + jnp.where(same, shifted, jnp.zeros_like(pv))
  nxt = _gath16(sk, jnp.minimum(iota + 1, 15))
  lastm = jnp.logical_or(iota == 15, sk != nxt)
  cur = plsc.load_gather(tab, [sk])
  nv = jnp.maximum(cur, pv) if op == "max" else cur + pv
  plsc.store_scatter(tab, [sk], nv, mask=lastm)


def _pre_body(feat_ref, wh_ref, wt_ref, we_ref, g_ref, b_ref,
              fh_ref, ft_ref, f0_ref):
  x = feat_ref[...]
  mu = jnp.mean(x, axis=-1, keepdims=True)
  var = jnp.mean(jnp.square(x - mu), axis=-1, keepdims=True)
  xn = (x - mu) * jax.lax.rsqrt(var + 1e-5) * g_ref[...] + b_ref[...]
  fh_ref[...] = jnp.dot(xn, wh_ref[...], preferred_element_type=jnp.float32)
  ft_ref[...] = jnp.dot(xn, wt_ref[...], preferred_element_type=jnp.float32)
  fe = jnp.dot(xn, we_ref[...], preferred_element_type=jnp.float32)
  for h in range(H):
    f0_ref[h] = fe[:, h * DH:(h + 1) * DH]


def _k_pre(feat, wh, wt, we, g, b):
  bn = 400
  return pl.pallas_call(
      _pre_body,
      grid=(N // bn,),
      in_specs=[pl.BlockSpec((bn, D), lambda i: (i, 0)),
                pl.BlockSpec((D, D), lambda i: (0, 0)),
                pl.BlockSpec((D, D), lambda i: (0, 0)),
                pl.BlockSpec((D, D), lambda i: (0, 0)),
                pl.BlockSpec((D,), lambda i: (0,)),
                pl.BlockSpec((D,), lambda i: (0,))],
      out_specs=[pl.BlockSpec((bn, D), lambda i: (i, 0)),
                 pl.BlockSpec((bn, D), lambda i: (i, 0)),
                 pl.BlockSpec((H, bn, DH), lambda i: (0, i, 0))],
      out_shape=[jax.ShapeDtypeStruct((N, D), jnp.float32),
                 jax.ShapeDtypeStruct((N, D), jnp.float32),
                 jax.ShapeDtypeStruct((H, N, DH), jnp.float32)],
  )(feat, wh, wt, we, g, b)


@functools.partial(
    pl.kernel, mesh=_MESH, compiler_params=_SC_PARAMS,
    out_type=(jax.ShapeDtypeStruct((NT, NT, CAPB), jnp.int32),
              jax.ShapeDtypeStruct((NT, NT), jnp.int32)),
    scratch_types=[pltpu.VMEM((ESH,), jnp.int32),
                   pltpu.VMEM((NT, CAPB), jnp.int32),
                   pltpu.VMEM((48,), jnp.int32)],
)
def _k_bin(dst_hbm, bins_o, cnt_o, shard_v, bins_v, cnt_v):
  tid = lax.axis_index("c") * 16 + lax.axis_index("s")
  pltpu.sync_copy(dst_hbm.at[pl.ds(tid * ESH, ESH)], shard_v)
  for k in range(2):
    cnt_v[pl.ds(16 * k, 16)] = jnp.zeros((16,), jnp.int32)
  iota = _i16()

  def body(j, _):
    d = shard_v[pl.ds(j * 16, 16)]
    r = d // RNG
    dl = d - r * RNG
    eid = tid * ESH + j * 16 + iota
    word = eid * 512 + dl
    sk, sv = plsc.sort_key_val(r, word)
    bnd = jnp.logical_or(iota == 0, _gath16(sk, jnp.maximum(iota - 1, 0)) != sk)
    first = plsc.cummax(jnp.where(bnd, iota, -1))
    rank = iota - first
    base = plsc.load_gather(cnt_v, [sk])
    pos = jnp.minimum(base + rank, CAPB - 1)
    plsc.store_scatter(bins_v, [sk, pos], sv)
    lastm = jnp.logical_or(iota == 15, _gath16(sk, jnp.minimum(iota + 1, 15)) != sk)
    plsc.store_scatter(cnt_v, [sk], jnp.minimum(base + rank + 1, CAPB), mask=lastm)
    return 0

  lax.fori_loop(0, ESH // 16, body, 0)
  pltpu.sync_copy(bins_v, bins_o.at[tid])
  pltpu.sync_copy(cnt_v.at[pl.ds(0, NT)], cnt_o.at[tid])


@functools.partial(
    pl.kernel, mesh=_MESH, compiler_params=_SC_PARAMS,
    out_type=(jax.ShapeDtypeStruct((E, D), jnp.float32),
              jax.ShapeDtypeStruct((E, D), jnp.float32)),
    scratch_types=[pltpu.VMEM((ESH,), jnp.int32),
                   pltpu.VMEM((ESH,), jnp.int32),
                   pltpu.VMEM((128, D), jnp.float32),
                   pltpu.VMEM((128, D), jnp.float32)],
)
def _k_gath(fh_hbm, ft_hbm, src_hbm, dst_hbm, fhs_o, fts_o,
            src_v, dst_v, hbuf, tbuf):
  tid = lax.axis_index("c") * 16 + lax.axis_index("s")
  base = tid * ESH
  pltpu.sync_copy(src_hbm.at[pl.ds(base, ESH)], src_v)
  pltpu.sync_copy(dst_hbm.at[pl.ds(base, ESH)], dst_v)

  def step(off, nb):
    pltpu.sync_copy(fh_hbm.at[src_v.at[pl.ds(off, nb)]], hbuf.at[pl.ds(0, nb)])
    pltpu.sync_copy(hbuf.at[pl.ds(0, nb)], fhs_o.at[pl.ds(base + off, nb)])
    pltpu.sync_copy(ft_hbm.at[dst_v.at[pl.ds(off, nb)]], tbuf.at[pl.ds(0, nb)])
    pltpu.sync_copy(tbuf.at[pl.ds(0, nb)], fts_o.at[pl.ds(base + off, nb)])

  def body(bi, _):
    step(bi * 128, 128)
    return 0

  lax.fori_loop(0, 78, body, 0)
  step(78 * 128, 16)


def _escore_body(fhs_ref, fts_ref, amat_ref, e_ref):
  s = fhs_ref[...] + fts_ref[...]
  l = jnp.maximum(s, SLOPE * s)
  res = jnp.dot(l, amat_ref[...], preferred_element_type=jnp.float32)
  e_ref[...] = res.T


def _k_escore(fhs, fts, amat):
  be = 2560
  return pl.pallas_call(
      _escore_body,
      grid=(E // be,),
      in_specs=[pl.BlockSpec((be, D), lambda i: (i, 0)),
                pl.BlockSpec((be, D), lambda i: (i, 0)),
                pl.BlockSpec((D, H), lambda i: (0, 0))],
      out_specs=pl.BlockSpec((H, be), lambda i: (0, i)),
      out_shape=jax.ShapeDtypeStruct((H, E), jnp.float32),
  )(fhs, fts, amat)


@functools.partial(
    pl.kernel, mesh=_MESH, compiler_params=_SC_PARAMS,
    out_type=(jax.ShapeDtypeStruct((H, NT, CAPS), jnp.int32),
              jax.ShapeDtypeStruct((H, NT, CAPS), jnp.int32),
              jax.ShapeDtypeStruct((H, NT, CAPS), jnp.float32),
              jax.ShapeDtypeStruct((NT, H), jnp.int32)),
    scratch_types=[pltpu.VMEM((1040,), jnp.int32),
                   pltpu.VMEM((CAPB,), jnp.int32),
                   pltpu.VMEM((CAPM + 16,), jnp.int32),
                   pltpu.VMEM((CAPM + 16,), jnp.int32),
                   pltpu.VMEM((CAPM + 16,), jnp.int32),
                   pltpu.VMEM((CAPM + 16,), jnp.int32),
                   pltpu.VMEM((CAPM + 16,), jnp.float32),
                   pltpu.VMEM((6, 320), jnp.float32),
                   pltpu.VMEM((320,), jnp.float32),
                   pltpu.VMEM((320,), jnp.float32),
                   pltpu.VMEM((320,), jnp.float32),
                   pltpu.VMEM((CAPS + 16,), jnp.int32),
                   pltpu.VMEM((CAPS + 16,), jnp.int32),
                   pltpu.VMEM((CAPS + 16,), jnp.float32),
                   pltpu.VMEM((16,), jnp.int32),
                   pltpu.SemaphoreType.DMA],
)
def _k_seg(e_hbm, src_hbm, bins_hbm, cnt_hbm,
           ssrc_o, sdl_o, sw_o, scnt_o,
           cnt_v, bin_v, eid_v, dl_v, src_v, idx_v, ecol_v,
           tabs_v, deg_v, ctab_v, den_v, stsrc_v, stdl_v, stw_v, c8_v, sem):
  rid = lax.axis_index("c") * 16 + lax.axis_index("s")
  iota = _i16()
  zf16 = jnp.zeros((16,), jnp.float32)
  neg16 = jnp.full((16,), NEG, jnp.float32)

  for t in range(NT):
    pltpu.sync_copy(cnt_hbm.at[t], cnt_v.at[pl.ds(t * NT, NT)])

  def zbody(j, _):
    eid_v[pl.ds(j * 16, 16)] = jnp.zeros((16,), jnp.int32)
    dl_v[pl.ds(j * 16, 16)] = jnp.zeros((16,), jnp.int32)
    return 0
  lax.fori_loop(0, (CAPM + 16) // 16, zbody, 0)

  def compact_t(t, m):
    nt = cnt_v[pl.ds(t * 32 + rid, 16)][0]
    pltpu.sync_copy(bins_hbm.at[t, rid], bin_v)

    def cbody(k, m):
      w = bin_v[pl.ds(k * 16, 16)]
      valid = k * 16 + iota < nt
      eid = jax.lax.shift_right_logical(w, 9)
      dl = jax.lax.bitwise_and(w, 511)
      plsc.store_compressed(eid_v.at[pl.ds(m, 16)], eid, mask=valid)
      plsc.store_compressed(dl_v.at[pl.ds(m, 16)], dl, mask=valid)
      return m + plsc.all_reduce_population_count(valid)[0]

    return lax.fori_loop(0, (nt + 15) // 16, cbody, m)

  m_tot = 0
  for t in range(NT):
    m_tot = compact_t(t, m_tot)
  m_tot = jnp.minimum(m_tot, CAPM)
  nch = (m_tot + 15) // 16

  for k in range(20):
    deg_v[pl.ds(k * 16, 16)] = zf16

  def degbody(j, _):
    dl = dl_v[pl.ds(j * 16, 16)]
    valid = j * 16 + iota < m_tot
    _seg_rmw(deg_v, jnp.where(valid, dl, 0),
             jnp.where(valid, 1.0, 0.0), "add")
    return 0
  lax.fori_loop(0, nch, degbody, 0)

  def logbody(k, _):
    dg = jnp.maximum(deg_v[pl.ds(k * 16, 16)], 1.0)
    bits = plsc.bitcast(dg, jnp.int32)
    ex = jax.lax.shift_right_logical(bits, 23) - 127
    mant = plsc.bitcast(jax.lax.bitwise_or(
        jax.lax.bitwise_and(bits, 0x007FFFFF), 0x3F800000), jnp.float32) - 1.0
    y = (ex.astype(jnp.float32) + mant) * 0.6931472
    for _ in range(3):
      y = y + dg * jnp.exp(-y) - 1.0
    ctab_v[pl.ds(k * 16, 16)] = y * (1.0 / DH)
    return 0
  lax.fori_loop(0, 20, logbody, 0)

  nb_m = (m_tot + 127) // 128

  def srcb(bi, _):
    pltpu.async_copy(src_hbm.at[eid_v.at[pl.ds(bi * 128, 128)]],
                     src_v.at[pl.ds(bi * 128, 128)], sem)
    return 0
  lax.fori_loop(0, nb_m, srcb, 0)

  def srcd(bi, _):
    pltpu.make_async_copy(src_hbm.at[pl.ds(0, 128)],
                          src_v.at[pl.ds(0, 128)], sem).wait()
    return 0
  lax.fori_loop(0, nb_m, srcd, 0)

  c8_v[pl.ds(0, 16)] = jnp.zeros((16,), jnp.int32)

  def head_body(h, _):
    def ib(j, _):
      idx_v[pl.ds(j * 16, 16)] = eid_v[pl.ds(j * 16, 16)] + h * E
      return 0
    lax.fori_loop(0, nch, ib, 0)

    def eb(bi, _):
      pltpu.async_copy(e_hbm.at[idx_v.at[pl.ds(bi * 128, 128)]],
                       ecol_v.at[pl.ds(bi * 128, 128)], sem)
      return 0
    lax.fori_loop(0, nb_m, eb, 0)

    def ebd(bi, _):
      pltpu.make_async_copy(e_hbm.at[pl.ds(0, 128)],
                            ecol_v.at[pl.ds(0, 128)], sem).wait()
      return 0
    lax.fori_loop(0, nb_m, ebd, 0)
    ecol_v[pl.ds(m_tot, 16)] = neg16

    for r in range(TOPK):
      def tinit(k, _, r=r):
        tabs_v[r, pl.ds(k * 16, 16)] = neg16
        return 0
      lax.fori_loop(0, 20, tinit, 0)

      def rbody(j, _, r=r):
        dl = dl_v[pl.ds(j * 16, 16)]
        b = ecol_v[pl.ds(j * 16, 16)]
        if r == 0:
          val = b
        else:
          prev = plsc.load_gather(tabs_v.at[r - 1], [dl])
          val = jnp.where(b < prev, b, NEG)
        _seg_rmw(tabs_v.at[r], dl, val, "max")
        return 0
      lax.fori_loop(0, nch, rbody, 0)

    def dinit(k, _):
      den_v[pl.ds(k * 16, 16)] = zf16
      return 0
    lax.fori_loop(0, 20, dinit, 0)

    def dbody(j, _):
      dl = dl_v[pl.ds(j * 16, 16)]
      b = ecol_v[pl.ds(j * 16, 16)]
      thr = plsc.load_gather(tabs_v.at[TOPK - 1], [dl])
      mx = plsc.load_gather(tabs_v.at[0], [dl])
      cc = plsc.load_gather(ctab_v, [dl])
      sel = jnp.logical_and(b >= thr, b > -1.0e38)
      v = jnp.where(sel, jnp.exp(cc * (b - mx)), 0.0)
      _seg_rmw(den_v, dl, v, "add")
      return 0
    lax.fori_loop(0, nch, dbody, 0)

    def ebody(j, cnt):
      dl = dl_v[pl.ds(j * 16, 16)]
      b = ecol_v[pl.ds(j * 16, 16)]
      thr = plsc.load_gather(tabs_v.at[TOPK - 1], [dl])
      mx = plsc.load_gather(tabs_v.at[0], [dl])
      cc = plsc.load_gather(ctab_v, [dl])
      dn = plsc.load_gather(den_v, [dl])
      sel = jnp.logical_and(b >= thr, b > -1.0e38)
      w = jnp.exp(cc * (b - mx)) / jnp.maximum(dn, 1e-38)
      sv = src_v[pl.ds(j * 16, 16)]
      cnt = jnp.minimum(cnt, CAPS)
      plsc.store_compressed(stsrc_v.at[pl.ds(cnt, 16)], sv, mask=sel)
      plsc.store_compressed(stdl_v.at[pl.ds(cnt, 16)], dl, mask=sel)
      plsc.store_compressed(stw_v.at[pl.ds(cnt, 16)], w, mask=sel)
      return cnt + plsc.all_reduce_population_count(sel)[0]
    cnt = lax.fori_loop(0, nch, ebody, 0)
    cnt = jnp.minimum(cnt, CAPS)

    pltpu.sync_copy(stsrc_v.at[pl.ds(0, CAPS)], ssrc_o.at[h, rid])
    pltpu.sync_copy(stdl_v.at[pl.ds(0, CAPS)], sdl_o.at[h, rid])
    pltpu.sync_copy(stw_v.at[pl.ds(0, CAPS)], sw_o.at[h, rid])
    c8_v[pl.ds(0, 16)] = jnp.where(iota == h, cnt, c8_v[pl.ds(0, 16)])
    return 0

  lax.fori_loop(0, H, head_body, 0)
  pltpu.sync_copy(c8_v.at[pl.ds(0, H)], scnt_o.at[rid])


def _hop_body(f_hbm, f0_hbm, ssrc_hbm, sdl_hbm, sw_hbm, scnt_hbm, out_o,
              agg_v, idx_v, gbuf, srcb_v, dlb_v, wb_v, c8_v,
              ab_v, fb_v, sem, last):
  rid = lax.axis_index("c") * 16 + lax.axis_index("s")
  lo = rid * RNG
  iota = _i16()

  def zb(j, _):
    agg_v[pl.ds(j * 16, 16)] = jnp.zeros((16,), jnp.float32)
    return 0
  lax.fori_loop(0, (H * RNG * DH) // 16, zb, 0)

  pltpu.sync_copy(scnt_hbm.at[rid], c8_v.at[pl.ds(0, H)])
  call = c8_v[...]

  for h in range(H):
    nsel = call[h]
    nbb = (nsel + 127) // 128
    pltpu.sync_copy(ssrc_hbm.at[h, rid], srcb_v)
    pltpu.sync_copy(sdl_hbm.at[h, rid], dlb_v)
    pltpu.sync_copy(sw_hbm.at[h, rid], wb_v)

    def ibody(sub, _, h=h):
      o = sub * 16
      valid = o + iota < nsel
      sv = jnp.where(valid, srcb_v[pl.ds(o, 16)], 0)
      idx_v[pl.ds(o, 16)] = sv + h * N
      return 0
    lax.fori_loop(0, nbb * 8, ibody, 0)

    def gfire(bi, _):
      pltpu.async_copy(f_hbm.at[idx_v.at[pl.ds(bi * 128, 128)]],
                       gbuf.at[pl.ds(bi * 128, 128)], sem)
      return 0
    lax.fori_loop(0, nbb, gfire, 0)

    def gdrain(bi, _):
      pltpu.make_async_copy(f_hbm.at[pl.ds(0, 128)],
                            gbuf.at[pl.ds(0, 128)], sem).wait()
      return 0
    lax.fori_loop(0, nbb, gdrain, 0)

    def ssub(sub, _, h=h):
      o = sub * 16
      valid = o + iota < nsel
      wv = jnp.where(valid, wb_v[pl.ds(o, 16)], 0.0)
      dv = jnp.where(valid, dlb_v[pl.ds(o, 16)], 0)
      base16 = dv * DH + h * (RNG * DH)
      for j in range(16):
        row = gbuf[o + j, :] * wv[j]
        b = base16[j]
        agg_v[pl.ds(b, 16)] = agg_v[pl.ds(b, 16)] + row
      return 0
    lax.fori_loop(0, nbb * 8, ssub, 0)

  for h in range(H):
    pltpu.sync_copy(f0_hbm.at[pl.ds(h * N + lo, RNG)], fb_v.at[pl.ds(0, RNG)])

    def blend(j, _, h=h):
      a = agg_v[pl.ds(h * (RNG * DH) + j * DH, 16)]
      ab_v[j, :] = (1.0 - ALPHA) * a + ALPHA * fb_v[j, :]
      return 0
    lax.fori_loop(0, RNG, blend, 0)
    if last:
      pltpu.sync_copy(ab_v.at[pl.ds(0, RNG)],
                      out_o.at[pl.ds(lo, RNG), pl.ds(h * DH, DH)])
    else:
      pltpu.sync_copy(ab_v.at[pl.ds(0, RNG)],
                      out_o.at[pl.ds(h * N + lo, RNG)])


def _make_hop(last):
  out_ty = (jax.ShapeDtypeStruct((N, D), jnp.float32) if last
            else jax.ShapeDtypeStruct((H * N, DH), jnp.float32))
  return functools.partial(
      pl.kernel, mesh=_MESH, compiler_params=_SC_PARAMS,
      out_type=out_ty,
      scratch_types=[pltpu.VMEM((H * RNG * DH,), jnp.float32),
                     pltpu.VMEM((CAPS,), jnp.int32),
                     pltpu.VMEM((CAPS, DH), jnp.float32),
                     pltpu.VMEM((CAPS,), jnp.int32),
                     pltpu.VMEM((CAPS,), jnp.int32),
                     pltpu.VMEM((CAPS,), jnp.float32),
                     pltpu.VMEM((16,), jnp.int32),
                     pltpu.VMEM((320, DH), jnp.float32),
                     pltpu.VMEM((320, DH), jnp.float32),
                     pltpu.SemaphoreType.DMA],
  )(functools.partial(_hop_body, last=last))


_k_hop_mid = _make_hop(False)
_k_hop_last = _make_hop(True)


def _ffn_body(f_ref, feat_ref, g_ref, b_ref, w1_ref, b1_ref, w2_ref, b2_ref,
              out_ref):
  rst = f_ref[...] + feat_ref[...]
  mu = jnp.mean(rst, axis=-1, keepdims=True)
  var = jnp.mean(jnp.square(rst - mu), axis=-1, keepdims=True)
  y = (rst - mu) * jax.lax.rsqrt(var + 1e-5) * g_ref[...] + b_ref[...]
  hdn = jnp.maximum(
      jnp.dot(y, w1_ref[...], preferred_element_type=jnp.float32)
      + b1_ref[...], 0.0)
  out_ref[...] = (jnp.dot(hdn, w2_ref[...], preferred_element_type=jnp.float32)
                  + b2_ref[...] + rst)


def _k_ffn(f2d, feat, g, b, w1, b1, w2, b2):
  bn = 400
  return pl.pallas_call(
      _ffn_body,
      grid=(N // bn,),
      in_specs=[pl.BlockSpec((bn, D), lambda i: (i, 0)),
                pl.BlockSpec((bn, D), lambda i: (i, 0)),
                pl.BlockSpec((D,), lambda i: (0,)),
                pl.BlockSpec((D,), lambda i: (0,)),
                pl.BlockSpec((D, 4 * D), lambda i: (0, 0)),
                pl.BlockSpec((4 * D,), lambda i: (0,)),
                pl.BlockSpec((4 * D, D), lambda i: (0, 0)),
                pl.BlockSpec((D,), lambda i: (0,))],
      out_specs=pl.BlockSpec((bn, D), lambda i: (i, 0)),
      out_shape=jax.ShapeDtypeStruct((N, D), jnp.float32),
  )(f2d, feat, g, b, w1, b1, w2, b2)


def kernel(feat, edge_index, W_head, W_tail, W_ent, attn,
           ln1_g, ln1_b, ln2_g, ln2_b, W_ff1, b_ff1, W_ff2, b_ff2):
  src = edge_index[0].astype(jnp.int32)
  dst = edge_index[1].astype(jnp.int32)
  attn2 = attn.reshape(H, DH)
  amat = (jnp.eye(H, dtype=jnp.float32)[:, None, :]
          * attn2[:, :, None]).reshape(D, H)

  fh, ft, f0 = _k_pre(feat, W_head, W_tail, W_ent, ln1_g, ln1_b)
  f0_flat = f0.reshape(H * N, DH)

  bins, bcnt = _k_bin(dst)
  fhs, fts = _k_gath(fh, ft, src, dst)
  e = _k_escore(fhs, fts, amat).reshape(H * E)

  ssrc, sdl, sw, scnt = _k_seg(e, src, bins, bcnt)

  f = f0_flat
  for _ in range(HOP - 1):
    f = _k_hop_mid(f, f0_flat, ssrc, sdl, sw, scnt)
  f2d = _k_hop_last(f, f0_flat, ssrc, sdl, sw, scnt)

  return _k_ffn(f2d, feat, ln2_g, ln2_b, W_ff1, b_ff1, W_ff2, b_ff2)


# speedup vs baseline: 78.7488x; 1.0435x over previous
import functools

import jax
import jax.numpy as jnp
from jax import lax
from jax.experimental import pallas as pl
from jax.experimental.pallas import tpu as pltpu
from jax.experimental.pallas import tpu_sc as plsc

N = 10000
E = 320000
D = 128
H = 8
DH = 16
HOP = 5
ALPHA = 0.1
TOPK = 5
SLOPE = 0.2

NT = 32
RNG = 313
ESH = E // NT
CAPB = 512
CAPM = 16384
CAPS = 2048
NEG = -3.0e38
BIG = 3.0e38

_SC_PARAMS = pltpu.CompilerParams(needs_layout_passes=False,
                                  use_tc_tiling_on_sc=False)
_MESH = plsc.VectorSubcoreMesh(core_axis_name="c", subcore_axis_name="s")
_GDN = jax.lax.GatherDimensionNumbers((), (0,), (0,))
_IN_BOUNDS = jax.lax.GatherScatterMode.PROMISE_IN_BOUNDS


def _i16():
  return lax.iota(jnp.int32, 16)


def _gath16(v, idx):
  return jax.lax.gather(v, idx[:, None], _GDN, (1,), mode=_IN_BOUNDS)


def _seg_rmw(tab, dl, val, op):
  iota = _i16()
  sk, sv = plsc.sort_key_val(dl, iota)
  pv = _gath16(val, sv)
  for s in (1, 2, 4, 8):
    src_lane = jnp.maximum(iota - s, 0)
    same = jnp.logical_and(_gath16(sk, src_lane) == sk, iota >= s)
    shifted = _gath16(pv, src_lane)
    if op == "max":
      pv = jnp.where(same, jnp.maximum(pv, shifted), pv)
    else:
      pv = pv + jnp.where(same, shifted, jnp.zeros_like(pv))
  nxt = _gath16(sk, jnp.minimum(iota + 1, 15))
  lastm = jnp.logical_or(iota == 15, sk != nxt)
  cur = plsc.load_gather(tab, [sk])
  nv = jnp.maximum(cur, pv) if op == "max" else cur + pv
  plsc.store_scatter(tab, [sk], nv, mask=lastm)


def _pre_body(feat_ref, wh_ref, wt_ref, we_ref, g_ref, b_ref,
              fh_ref, ft_ref, f0_ref):
  x = feat_ref[...]
  mu = jnp.mean(x, axis=-1, keepdims=True)
  var = jnp.mean(jnp.square(x - mu), axis=-1, keepdims=True)
  xn = (x - mu) * jax.lax.rsqrt(var + 1e-5) * g_ref[...] + b_ref[...]
  fh_ref[...] = jnp.dot(xn, wh_ref[...], preferred_element_type=jnp.float32)
  ft_ref[...] = jnp.dot(xn, wt_ref[...], preferred_element_type=jnp.float32)
  fe = jnp.dot(xn, we_ref[...], preferred_element_type=jnp.float32)
  for h in range(H):
    f0_ref[h] = fe[:, h * DH:(h + 1) * DH]


def _k_pre(feat, wh, wt, we, g, b):
  bn = 400
  return pl.pallas_call(
      _pre_body,
      grid=(N // bn,),
      in_specs=[pl.BlockSpec((bn, D), lambda i: (i, 0)),
                pl.BlockSpec((D, D), lambda i: (0, 0)),
                pl.BlockSpec((D, D), lambda i: (0, 0)),
                pl.BlockSpec((D, D), lambda i: (0, 0)),
                pl.BlockSpec((D,), lambda i: (0,)),
                pl.BlockSpec((D,), lambda i: (0,))],
      out_specs=[pl.BlockSpec((bn, D), lambda i: (i, 0)),
                 pl.BlockSpec((bn, D), lambda i: (i, 0)),
                 pl.BlockSpec((H, bn, DH), lambda i: (0, i, 0))],
      out_shape=[jax.ShapeDtypeStruct((N, D), jnp.float32),
                 jax.ShapeDtypeStruct((N, D), jnp.float32),
                 jax.ShapeDtypeStruct((H, N, DH), jnp.float32)],
  )(feat, wh, wt, we, g, b)


@functools.partial(
    pl.kernel, mesh=_MESH, compiler_params=_SC_PARAMS,
    out_type=(jax.ShapeDtypeStruct((NT, NT, CAPB), jnp.int32),
              jax.ShapeDtypeStruct((NT, NT), jnp.int32)),
    scratch_types=[pltpu.VMEM((ESH,), jnp.int32),
                   pltpu.VMEM((NT, CAPB), jnp.int32),
                   pltpu.VMEM((48,), jnp.int32)],
)
def _k_bin(dst_hbm, bins_o, cnt_o, shard_v, bins_v, cnt_v):
  tid = lax.axis_index("c") * 16 + lax.axis_index("s")
  pltpu.sync_copy(dst_hbm.at[pl.ds(tid * ESH, ESH)], shard_v)
  for k in range(2):
    cnt_v[pl.ds(16 * k, 16)] = jnp.zeros((16,), jnp.int32)
  iota = _i16()

  def body(j, _):
    d = shard_v[pl.ds(j * 16, 16)]
    r = d // RNG
    dl = d - r * RNG
    eid = tid * ESH + j * 16 + iota
    word = eid * 512 + dl
    sk, sv = plsc.sort_key_val(r, word)
    bnd = jnp.logical_or(iota == 0, _gath16(sk, jnp.maximum(iota - 1, 0)) != sk)
    first = plsc.cummax(jnp.where(bnd, iota, -1))
    rank = iota - first
    base = plsc.load_gather(cnt_v, [sk])
    pos = jnp.minimum(base + rank, CAPB - 1)
    plsc.store_scatter(bins_v, [sk, pos], sv)
    lastm = jnp.logical_or(iota == 15, _gath16(sk, jnp.minimum(iota + 1, 15)) != sk)
    plsc.store_scatter(cnt_v, [sk], jnp.minimum(base + rank + 1, CAPB), mask=lastm)
    return 0

  lax.fori_loop(0, ESH // 16, body, 0)
  pltpu.sync_copy(bins_v, bins_o.at[tid])
  pltpu.sync_copy(cnt_v.at[pl.ds(0, NT)], cnt_o.at[tid])


@functools.partial(
    pl.kernel, mesh=_MESH, compiler_params=_SC_PARAMS,
    out_type=(jax.ShapeDtypeStruct((E, D), jnp.float32),
              jax.ShapeDtypeStruct((E, D), jnp.float32)),
    scratch_types=[pltpu.VMEM((ESH,), jnp.int32),
                   pltpu.VMEM((ESH,), jnp.int32),
                   pltpu.VMEM((2, 128, D), jnp.float32),
                   pltpu.VMEM((2, 128, D), jnp.float32),
                   pltpu.SemaphoreType.DMA,
                   pltpu.SemaphoreType.DMA,
                   pltpu.SemaphoreType.DMA,
                   pltpu.SemaphoreType.DMA],
)
def _k_gath(fh_hbm, ft_hbm, src_hbm, dst_hbm, fhs_o, fts_o,
            src_v, dst_v, hbuf, tbuf, sgh, swh, sgt, swt):
  tid = lax.axis_index("c") * 16 + lax.axis_index("s")
  base = tid * ESH
  pltpu.sync_copy(src_hbm.at[pl.ds(base, ESH)], src_v)
  pltpu.sync_copy(dst_hbm.at[pl.ds(base, ESH)], dst_v)

  nb_full = ESH // 128
  sizes = [128] * nb_full + [16]
  gh = {}
  wh = {}
  gt = {}
  wt = {}
  for b in range(len(sizes) + 1):
    if b < len(sizes):
      if b >= 2:
        wh[b - 2].wait()
        wt[b - 2].wait()
      off = b * 128
      nb = sizes[b]
      gh[b] = pltpu.async_copy(fh_hbm.at[src_v.at[pl.ds(off, nb)]],
                               hbuf.at[b % 2, pl.ds(0, nb)], sgh)
      gt[b] = pltpu.async_copy(ft_hbm.at[dst_v.at[pl.ds(off, nb)]],
                               tbuf.at[b % 2, pl.ds(0, nb)], sgt)
    if b >= 1:
      p = b - 1
      off = p * 128
      nb = sizes[p]
      gh[p].wait()
      wh[p] = pltpu.async_copy(hbuf.at[p % 2, pl.ds(0, nb)],
                               fhs_o.at[pl.ds(base + off, nb)], swh)
      gt[p].wait()
      wt[p] = pltpu.async_copy(tbuf.at[p % 2, pl.ds(0, nb)],
                               fts_o.at[pl.ds(base + off, nb)], swt)
  wh[len(sizes) - 1].wait()
  wt[len(sizes) - 1].wait()
  wh[len(sizes) - 2].wait()
  wt[len(sizes) - 2].wait()


def _escore_body(fhs_ref, fts_ref, amat_ref, e_ref):
  s = fhs_ref[...] + fts_ref[...]
  l = jnp.maximum(s, SLOPE * s)
  res = jnp.dot(l, amat_ref[...], preferred_element_type=jnp.float32)
  e_ref[...] = res.T


def _k_escore(fhs, fts, amat):
  be = 2560
  return pl.pallas_call(
      _escore_body,
      grid=(E // be,),
      in_specs=[pl.BlockSpec((be, D), lambda i: (i, 0)),
                pl.BlockSpec((be, D), lambda i: (i, 0)),
                pl.BlockSpec((D, H), lambda i: (0, 0))],
      out_specs=pl.BlockSpec((H, be), lambda i: (0, i)),
      out_shape=jax.ShapeDtypeStruct((H, E), jnp.float32),
  )(fhs, fts, amat)


@functools.partial(
    pl.kernel, mesh=_MESH, compiler_params=_SC_PARAMS,
    out_type=(jax.ShapeDtypeStruct((H, NT, CAPS), jnp.int32),
              jax.ShapeDtypeStruct((H, NT, CAPS), jnp.int32),
              jax.ShapeDtypeStruct((H, NT, CAPS), jnp.float32),
              jax.ShapeDtypeStruct((NT, H), jnp.int32)),
    scratch_types=[pltpu.VMEM((1040,), jnp.int32),
                   pltpu.VMEM((CAPB,), jnp.int32),
                   pltpu.VMEM((CAPM + 16,), jnp.int32),
                   pltpu.VMEM((CAPM + 16,), jnp.int32),
                   pltpu.VMEM((CAPM + 16,), jnp.int32),
                   pltpu.VMEM((CAPM + 16,), jnp.int32),
                   pltpu.VMEM((CAPM + 16,), jnp.float32),
                   pltpu.VMEM((6, 320), jnp.float32),
                   pltpu.VMEM((320,), jnp.float32),
                   pltpu.VMEM((320,), jnp.float32),
                   pltpu.VMEM((320,), jnp.float32),
                   pltpu.VMEM((CAPS + 16,), jnp.int32),
                   pltpu.VMEM((CAPS + 16,), jnp.int32),
                   pltpu.VMEM((CAPS + 16,), jnp.float32),
                   pltpu.VMEM((16,), jnp.int32),
                   pltpu.SemaphoreType.DMA],
)
def _k_seg(e_hbm, src_hbm, bins_hbm, cnt_hbm,
           ssrc_o, sdl_o, sw_o, scnt_o,
           cnt_v, bin_v, eid_v, dl_v, src_v, idx_v, ecol_v,
           tabs_v, deg_v, ctab_v, den_v, stsrc_v, stdl_v, stw_v, c8_v, sem):
  rid = lax.axis_index("c") * 16 + lax.axis_index("s")
  iota = _i16()
  zf16 = jnp.zeros((16,), jnp.float32)
  neg16 = jnp.full((16,), NEG, jnp.float32)

  for t in range(NT):
    pltpu.sync_copy(cnt_hbm.at[t], cnt_v.at[pl.ds(t * NT, NT)])

  def zbody(j, _):
    eid_v[pl.ds(j * 16, 16)] = jnp.zeros((16,), jnp.int32)
    dl_v[pl.ds(j * 16, 16)] = jnp.zeros((16,), jnp.int32)
    return 0
  lax.fori_loop(0, (CAPM + 16) // 16, zbody, 0)

  def compact_t(t, m):
    nt = cnt_v[pl.ds(t * 32 + rid, 16)][0]
    pltpu.sync_copy(bins_hbm.at[t, rid], bin_v)

    def cbody(k, m):
      w = bin_v[pl.ds(k * 16, 16)]
      valid = k * 16 + iota < nt
      eid = jax.lax.shift_right_logical(w, 9)
      dl = jax.lax.bitwise_and(w, 511)
      plsc.store_compressed(eid_v.at[pl.ds(m, 16)], eid, mask=valid)
      plsc.store_compressed(dl_v.at[pl.ds(m, 16)], dl, mask=valid)
      return m + plsc.all_reduce_population_count(valid)[0]

    return lax.fori_loop(0, (nt + 15) // 16, cbody, m)

  m_tot = 0
  for t in range(NT):
    m_tot = compact_t(t, m_tot)
  m_tot = jnp.minimum(m_tot, CAPM)
  nch = (m_tot + 15) // 16

  for k in range(20):
    deg_v[pl.ds(k * 16, 16)] = zf16

  def degbody(j, _):
    dl = dl_v[pl.ds(j * 16, 16)]
    valid = j * 16 + iota < m_tot
    _seg_rmw(deg_v, jnp.where(valid, dl, 0),
             jnp.where(valid, 1.0, 0.0), "add")
    return 0
  lax.fori_loop(0, nch, degbody, 0)

  def logbody(k, _):
    dg = jnp.maximum(deg_v[pl.ds(k * 16, 16)], 1.0)
    bits = plsc.bitcast(dg, jnp.int32)
    ex = jax.lax.shift_right_logical(bits, 23) - 127
    mant = plsc.bitcast(jax.lax.bitwise_or(
        jax.lax.bitwise_and(bits, 0x007FFFFF), 0x3F800000), jnp.float32) - 1.0
    y = (ex.astype(jnp.float32) + mant) * 0.6931472
    for _ in range(3):
      y = y + dg * jnp.exp(-y) - 1.0
    ctab_v[pl.ds(k * 16, 16)] = y * (1.0 / DH)
    return 0
  lax.fori_loop(0, 20, logbody, 0)

  nb_m = (m_tot + 127) // 128

  def srcb(bi, _):
    pltpu.async_copy(src_hbm.at[eid_v.at[pl.ds(bi * 128, 128)]],
                     src_v.at[pl.ds(bi * 128, 128)], sem)
    return 0
  lax.fori_loop(0, nb_m, srcb, 0)

  def srcd(bi, _):
    pltpu.make_async_copy(src_hbm.at[pl.ds(0, 128)],
                          src_v.at[pl.ds(0, 128)], sem).wait()
    return 0
  lax.fori_loop(0, nb_m, srcd, 0)

  c8_v[pl.ds(0, 16)] = jnp.zeros((16,), jnp.int32)

  def head_body(h, _):
    def ib(j, _):
      idx_v[pl.ds(j * 16, 16)] = eid_v[pl.ds(j * 16, 16)] + h * E
      return 0
    lax.fori_loop(0, nch, ib, 0)

    def eb(bi, _):
      pltpu.async_copy(e_hbm.at[idx_v.at[pl.ds(bi * 128, 128)]],
                       ecol_v.at[pl.ds(bi * 128, 128)], sem)
      return 0
    lax.fori_loop(0, nb_m, eb, 0)

    def ebd(bi, _):
      pltpu.make_async_copy(e_hbm.at[pl.ds(0, 128)],
                            ecol_v.at[pl.ds(0, 128)], sem).wait()
      return 0
    lax.fori_loop(0, nb_m, ebd, 0)
    ecol_v[pl.ds(m_tot, 16)] = neg16

    for r in range(TOPK):
      def tinit(k, _, r=r):
        tabs_v[r, pl.ds(k * 16, 16)] = neg16
        return 0
      lax.fori_loop(0, 20, tinit, 0)

      def rbody(j, _, r=r):
        dl = dl_v[pl.ds(j * 16, 16)]
        b = ecol_v[pl.ds(j * 16, 16)]
        if r == 0:
          val = b
        else:
          prev = plsc.load_gather(tabs_v.at[r - 1], [dl])
          val = jnp.where(b < prev, b, NEG)
        _seg_rmw(tabs_v.at[r], dl, val, "max")
        return 0
      lax.fori_loop(0, nch, rbody, 0)

    def dinit(k, _):
      den_v[pl.ds(k * 16, 16)] = zf16
      return 0
    lax.fori_loop(0, 20, dinit, 0)

    def dbody(j, _):
      dl = dl_v[pl.ds(j * 16, 16)]
      b = ecol_v[pl.ds(j * 16, 16)]
      thr = plsc.load_gather(tabs_v.at[TOPK - 1], [dl])
      mx = plsc.load_gather(tabs_v.at[0], [dl])
      cc = plsc.load_gather(ctab_v, [dl])
      sel = jnp.logical_and(b >= thr, b > -1.0e38)
      v = jnp.where(sel, jnp.exp(cc * (b - mx)), 0.0)
      _seg_rmw(den_v, dl, v, "add")
      return 0
    lax.fori_loop(0, nch, dbody, 0)

    def ebody(j, cnt):
      dl = dl_v[pl.ds(j * 16, 16)]
      b = ecol_v[pl.ds(j * 16, 16)]
      thr = plsc.load_gather(tabs_v.at[TOPK - 1], [dl])
      mx = plsc.load_gather(tabs_v.at[0], [dl])
      cc = plsc.load_gather(ctab_v, [dl])
      dn = plsc.load_gather(den_v, [dl])
      sel = jnp.logical_and(b >= thr, b > -1.0e38)
      w = jnp.exp(cc * (b - mx)) / jnp.maximum(dn, 1e-38)
      sv = src_v[pl.ds(j * 16, 16)]
      cnt = jnp.minimum(cnt, CAPS)
      plsc.store_compressed(stsrc_v.at[pl.ds(cnt, 16)], sv, mask=sel)
      plsc.store_compressed(stdl_v.at[pl.ds(cnt, 16)], dl, mask=sel)
      plsc.store_compressed(stw_v.at[pl.ds(cnt, 16)], w, mask=sel)
      return cnt + plsc.all_reduce_population_count(sel)[0]
    cnt = lax.fori_loop(0, nch, ebody, 0)
    cnt = jnp.minimum(cnt, CAPS)

    pltpu.sync_copy(stsrc_v.at[pl.ds(0, CAPS)], ssrc_o.at[h, rid])
    pltpu.sync_copy(stdl_v.at[pl.ds(0, CAPS)], sdl_o.at[h, rid])
    pltpu.sync_copy(stw_v.at[pl.ds(0, CAPS)], sw_o.at[h, rid])
    c8_v[pl.ds(0, 16)] = jnp.where(iota == h, cnt, c8_v[pl.ds(0, 16)])
    return 0

  lax.fori_loop(0, H, head_body, 0)
  pltpu.sync_copy(c8_v.at[pl.ds(0, H)], scnt_o.at[rid])


def _hop_body(f_hbm, f0_hbm, ssrc_hbm, sdl_hbm, sw_hbm, scnt_hbm, out_o,
              agg_v, idx_v, gbuf, srcb_v, dlb_v, wb_v, c8_v,
              ab_v, fb_v, sem, last):
  rid = lax.axis_index("c") * 16 + lax.axis_index("s")
  lo = rid * RNG
  iota = _i16()

  def zb(j, _):
    agg_v[pl.ds(j * 16, 16)] = jnp.zeros((16,), jnp.float32)
    return 0
  lax.fori_loop(0, (H * RNG * DH) // 16, zb, 0)

  pltpu.sync_copy(scnt_hbm.at[rid], c8_v.at[pl.ds(0, H)])
  call = c8_v[...]

  for h in range(H):
    nsel = call[h]
    nbb = (nsel + 127) // 128
    pltpu.sync_copy(ssrc_hbm.at[h, rid], srcb_v)
    pltpu.sync_copy(sdl_hbm.at[h, rid], dlb_v)
    pltpu.sync_copy(sw_hbm.at[h, rid], wb_v)

    def ibody(sub, _, h=h):
      o = sub * 16
      valid = o + iota < nsel
      sv = jnp.where(valid, srcb_v[pl.ds(o, 16)], 0)
      idx_v[pl.ds(o, 16)] = sv + h * N
      return 0
    lax.fori_loop(0, nbb * 8, ibody, 0)

    def gfire(bi, _):
      pltpu.async_copy(f_hbm.at[idx_v.at[pl.ds(bi * 128, 128)]],
                       gbuf.at[pl.ds(bi * 128, 128)], sem)
      return 0
    lax.fori_loop(0, nbb, gfire, 0)

    def gdrain(bi, _):
      pltpu.make_async_copy(f_hbm.at[pl.ds(0, 128)],
                            gbuf.at[pl.ds(0, 128)], sem).wait()
      return 0
    lax.fori_loop(0, nbb, gdrain, 0)

    def ssub(sub, _, h=h):
      o = sub * 16
      valid = o + iota < nsel
      wv = jnp.where(valid, wb_v[pl.ds(o, 16)], 0.0)
      dv = jnp.where(valid, dlb_v[pl.ds(o, 16)], 0)
      base16 = dv * DH + h * (RNG * DH)
      for j in range(16):
        row = gbuf[o + j, :] * wv[j]
        b = base16[j]
        agg_v[pl.ds(b, 16)] = agg_v[pl.ds(b, 16)] + row
      return 0
    lax.fori_loop(0, nbb * 8, ssub, 0)

  for h in range(H):
    pltpu.sync_copy(f0_hbm.at[pl.ds(h * N + lo, RNG)], fb_v.at[pl.ds(0, RNG)])

    def blend(j, _, h=h):
      a = agg_v[pl.ds(h * (RNG * DH) + j * DH, 16)]
      ab_v[j, :] = (1.0 - ALPHA) * a + ALPHA * fb_v[j, :]
      return 0
    lax.fori_loop(0, RNG, blend, 0)
    if last:
      pltpu.sync_copy(ab_v.at[pl.ds(0, RNG)],
                      out_o.at[pl.ds(lo, RNG), pl.ds(h * DH, DH)])
    else:
      pltpu.sync_copy(ab_v.at[pl.ds(0, RNG)],
                      out_o.at[pl.ds(h * N + lo, RNG)])


def _make_hop(last):
  out_ty = (jax.ShapeDtypeStruct((N, D), jnp.float32) if last
            else jax.ShapeDtypeStruct((H * N, DH), jnp.float32))
  return functools.partial(
      pl.kernel, mesh=_MESH, compiler_params=_SC_PARAMS,
      out_type=out_ty,
      scratch_types=[pltpu.VMEM((H * RNG * DH,), jnp.float32),
                     pltpu.VMEM((CAPS,), jnp.int32),
                     pltpu.VMEM((CAPS, DH), jnp.float32),
                     pltpu.VMEM((CAPS,), jnp.int32),
                     pltpu.VMEM((CAPS,), jnp.int32),
                     pltpu.VMEM((CAPS,), jnp.float32),
                     pltpu.VMEM((16,), jnp.int32),
                     pltpu.VMEM((320, DH), jnp.float32),
                     pltpu.VMEM((320, DH), jnp.float32),
                     pltpu.SemaphoreType.DMA],
  )(functools.partial(_hop_body, last=last))


_k_hop_mid = _make_hop(False)
_k_hop_last = _make_hop(True)


def _ffn_body(f_ref, feat_ref, g_ref, b_ref, w1_ref, b1_ref, w2_ref, b2_ref,
              out_ref):
  rst = f_ref[...] + feat_ref[...]
  mu = jnp.mean(rst, axis=-1, keepdims=True)
  var = jnp.mean(jnp.square(rst - mu), axis=-1, keepdims=True)
  y = (rst - mu) * jax.lax.rsqrt(var + 1e-5) * g_ref[...] + b_ref[...]
  hdn = jnp.maximum(
      jnp.dot(y, w1_ref[...], preferred_element_type=jnp.float32)
      + b1_ref[...], 0.0)
  out_ref[...] = (jnp.dot(hdn, w2_ref[...], preferred_element_type=jnp.float32)
                  + b2_ref[...] + rst)


def _k_ffn(f2d, feat, g, b, w1, b1, w2, b2):
  bn = 400
  return pl.pallas_call(
      _ffn_body,
      grid=(N // bn,),
      in_specs=[pl.BlockSpec((bn, D), lambda i: (i, 0)),
                pl.BlockSpec((bn, D), lambda i: (i, 0)),
                pl.BlockSpec((D,), lambda i: (0,)),
                pl.BlockSpec((D,), lambda i: (0,)),
                pl.BlockSpec((D, 4 * D), lambda i: (0, 0)),
                pl.BlockSpec((4 * D,), lambda i: (0,)),
                pl.BlockSpec((4 * D, D), lambda i: (0, 0)),
                pl.BlockSpec((D,), lambda i: (0,))],
      out_specs=pl.BlockSpec((bn, D), lambda i: (i, 0)),
      out_shape=jax.ShapeDtypeStruct((N, D), jnp.float32),
  )(f2d, feat, g, b, w1, b1, w2, b2)


def kernel(feat, edge_index, W_head, W_tail, W_ent, attn,
           ln1_g, ln1_b, ln2_g, ln2_b, W_ff1, b_ff1, W_ff2, b_ff2):
  src = edge_index[0].astype(jnp.int32)
  dst = edge_index[1].astype(jnp.int32)
  attn2 = attn.reshape(H, DH)
  amat = (jnp.eye(H, dtype=jnp.float32)[:, None, :]
          * attn2[:, :, None]).reshape(D, H)

  fh, ft, f0 = _k_pre(feat, W_head, W_tail, W_ent, ln1_g, ln1_b)
  f0_flat = f0.reshape(H * N, DH)

  bins, bcnt = _k_bin(dst)
  fhs, fts = _k_gath(fh, ft, src, dst)
  e = _k_escore(fhs, fts, amat).reshape(H * E)

  ssrc, sdl, sw, scnt = _k_seg(e, src, bins, bcnt)

  f = f0_flat
  for _ in range(HOP - 1):
    f = _k_hop_mid(f, f0_flat, ssrc, sdl, sw, scnt)
  f2d = _k_hop_last(f, f0_flat, ssrc, sdl, sw, scnt)

  return _k_ffn(f2d, feat, ln2_g, ln2_b, W_ff1, b_ff1, W_ff2, b_ff2)


# speedup vs baseline: 81.1391x; 1.0304x over previous
import functools

import jax
import jax.numpy as jnp
from jax import lax
from jax.experimental import pallas as pl
from jax.experimental.pallas import tpu as pltpu
from jax.experimental.pallas import tpu_sc as plsc

N = 10000
E = 320000
D = 128
H = 8
DH = 16
HOP = 5
ALPHA = 0.1
TOPK = 5
SLOPE = 0.2

NT = 32
RNG = 313
ESH = E // NT
CAPB = 512
CAPM = 16384
CAPS = 2048
NEG = -3.0e38
BIG = 3.0e38

_SC_PARAMS = pltpu.CompilerParams(needs_layout_passes=False,
                                  use_tc_tiling_on_sc=False)
_MESH = plsc.VectorSubcoreMesh(core_axis_name="c", subcore_axis_name="s")
_GDN = jax.lax.GatherDimensionNumbers((), (0,), (0,))
_IN_BOUNDS = jax.lax.GatherScatterMode.PROMISE_IN_BOUNDS


def _i16():
  return lax.iota(jnp.int32, 16)


def _gath16(v, idx):
  return jax.lax.gather(v, idx[:, None], _GDN, (1,), mode=_IN_BOUNDS)


def _prefix_combine(pk, pv, op):
  iota = _i16()
  for bi, s in enumerate((1, 2, 4, 8)):
    same = jax.lax.bitwise_and(
        jax.lax.shift_right_logical(pk, 13 + bi), 1) == 1
    shifted = _gath16(pv, jnp.maximum(iota - s, 0))
    if op == "max":
      pv = jnp.where(same, jnp.maximum(pv, shifted), pv)
    else:
      pv = pv + jnp.where(same, shifted, jnp.zeros_like(pv))
  return pv


def _seg_rmw_pre(tab, pk, pv, op):
  sk = jax.lax.bitwise_and(jax.lax.shift_right_logical(pk, 4), 511)
  pv = _prefix_combine(pk, pv, op)
  lastm = jax.lax.bitwise_and(jax.lax.shift_right_logical(pk, 17), 1) == 1
  cur = plsc.load_gather(tab, [sk])
  nv = jnp.maximum(cur, pv) if op == "max" else cur + pv
  plsc.store_scatter(tab, [sk], nv, mask=lastm)


def _seg_rmw(tab, dl, val, op):
  iota = _i16()
  sk, sv = plsc.sort_key_val(dl, iota)
  pv = _gath16(val, sv)
  for s in (1, 2, 4, 8):
    src_lane = jnp.maximum(iota - s, 0)
    same = jnp.logical_and(_gath16(sk, src_lane) == sk, iota >= s)
    shifted = _gath16(pv, src_lane)
    if op == "max":
      pv = jnp.where(same, jnp.maximum(pv, shifted), pv)
    else:
      pv = pv + jnp.where(same, shifted, jnp.zeros_like(pv))
  nxt = _gath16(sk, jnp.minimum(iota + 1, 15))
  lastm = jnp.logical_or(iota == 15, sk != nxt)
  cur = plsc.load_gather(tab, [sk])
  nv = jnp.maximum(cur, pv) if op == "max" else cur + pv
  plsc.store_scatter(tab, [sk], nv, mask=lastm)


def _pre_body(feat_ref, wh_ref, wt_ref, we_ref, g_ref, b_ref,
              fh_ref, ft_ref, f0_ref):
  x = feat_ref[...]
  mu = jnp.mean(x, axis=-1, keepdims=True)
  var = jnp.mean(jnp.square(x - mu), axis=-1, keepdims=True)
  xn = (x - mu) * jax.lax.rsqrt(var + 1e-5) * g_ref[...] + b_ref[...]
  fh_ref[...] = jnp.dot(xn, wh_ref[...], preferred_element_type=jnp.float32)
  ft_ref[...] = jnp.dot(xn, wt_ref[...], preferred_element_type=jnp.float32)
  fe = jnp.dot(xn, we_ref[...], preferred_element_type=jnp.float32)
  for h in range(H):
    f0_ref[h] = fe[:, h * DH:(h + 1) * DH]


def _k_pre(feat, wh, wt, we, g, b):
  bn = 400
  return pl.pallas_call(
      _pre_body,
      grid=(N // bn,),
      in_specs=[pl.BlockSpec((bn, D), lambda i: (i, 0)),
                pl.BlockSpec((D, D), lambda i: (0, 0)),
                pl.BlockSpec((D, D), lambda i: (0, 0)),
                pl.BlockSpec((D, D), lambda i: (0, 0)),
                pl.BlockSpec((D,), lambda i: (0,)),
                pl.BlockSpec((D,), lambda i: (0,))],
      out_specs=[pl.BlockSpec((bn, D), lambda i: (i, 0)),
                 pl.BlockSpec((bn, D), lambda i: (i, 0)),
                 pl.BlockSpec((H, bn, DH), lambda i: (0, i, 0))],
      out_shape=[jax.ShapeDtypeStruct((N, D), jnp.float32),
                 jax.ShapeDtypeStruct((N, D), jnp.float32),
                 jax.ShapeDtypeStruct((H, N, DH), jnp.float32)],
  )(feat, wh, wt, we, g, b)


@functools.partial(
    pl.kernel, mesh=_MESH, compiler_params=_SC_PARAMS,
    out_type=(jax.ShapeDtypeStruct((NT, NT, CAPB), jnp.int32),
              jax.ShapeDtypeStruct((NT, NT), jnp.int32)),
    scratch_types=[pltpu.VMEM((ESH,), jnp.int32),
                   pltpu.VMEM((NT, CAPB), jnp.int32),
                   pltpu.VMEM((48,), jnp.int32)],
)
def _k_bin(dst_hbm, bins_o, cnt_o, shard_v, bins_v, cnt_v):
  tid = lax.axis_index("c") * 16 + lax.axis_index("s")
  pltpu.sync_copy(dst_hbm.at[pl.ds(tid * ESH, ESH)], shard_v)
  for k in range(2):
    cnt_v[pl.ds(16 * k, 16)] = jnp.zeros((16,), jnp.int32)
  iota = _i16()

  def body(j, _):
    d = shard_v[pl.ds(j * 16, 16)]
    r = d // RNG
    dl = d - r * RNG
    eid = tid * ESH + j * 16 + iota
    word = eid * 512 + dl
    sk, sv = plsc.sort_key_val(r, word)
    bnd = jnp.logical_or(iota == 0, _gath16(sk, jnp.maximum(iota - 1, 0)) != sk)
    first = plsc.cummax(jnp.where(bnd, iota, -1))
    rank = iota - first
    base = plsc.load_gather(cnt_v, [sk])
    pos = jnp.minimum(base + rank, CAPB - 1)
    plsc.store_scatter(bins_v, [sk, pos], sv)
    lastm = jnp.logical_or(iota == 15, _gath16(sk, jnp.minimum(iota + 1, 15)) != sk)
    plsc.store_scatter(cnt_v, [sk], jnp.minimum(base + rank + 1, CAPB), mask=lastm)
    return 0

  lax.fori_loop(0, ESH // 16, body, 0)
  pltpu.sync_copy(bins_v, bins_o.at[tid])
  pltpu.sync_copy(cnt_v.at[pl.ds(0, NT)], cnt_o.at[tid])


@functools.partial(
    pl.kernel, mesh=_MESH, compiler_params=_SC_PARAMS,
    out_type=(jax.ShapeDtypeStruct((E, D), jnp.float32),
              jax.ShapeDtypeStruct((E, D), jnp.float32)),
    scratch_types=[pltpu.VMEM((ESH,), jnp.int32),
                   pltpu.VMEM((ESH,), jnp.int32),
                   pltpu.VMEM((2, 128, D), jnp.float32),
                   pltpu.VMEM((2, 128, D), jnp.float32),
                   pltpu.SemaphoreType.DMA,
                   pltpu.SemaphoreType.DMA,
                   pltpu.SemaphoreType.DMA,
                   pltpu.SemaphoreType.DMA],
)
def _k_gath(fh_hbm, ft_hbm, src_hbm, dst_hbm, fhs_o, fts_o,
            src_v, dst_v, hbuf, tbuf, sgh, swh, sgt, swt):
  tid = lax.axis_index("c") * 16 + lax.axis_index("s")
  base = tid * ESH
  pltpu.sync_copy(src_hbm.at[pl.ds(base, ESH)], src_v)
  pltpu.sync_copy(dst_hbm.at[pl.ds(base, ESH)], dst_v)

  nb_full = ESH // 128
  sizes = [128] * nb_full + [16]
  gh = {}
  wh = {}
  gt = {}
  wt = {}
  for b in range(len(sizes) + 1):
    if b < len(sizes):
      if b >= 2:
        wh[b - 2].wait()
        wt[b - 2].wait()
      off = b * 128
      nb = sizes[b]
      gh[b] = pltpu.async_copy(fh_hbm.at[src_v.at[pl.ds(off, nb)]],
                               hbuf.at[b % 2, pl.ds(0, nb)], sgh)
      gt[b] = pltpu.async_copy(ft_hbm.at[dst_v.at[pl.ds(off, nb)]],
                               tbuf.at[b % 2, pl.ds(0, nb)], sgt)
    if b >= 1:
      p = b - 1
      off = p * 128
      nb = sizes[p]
      gh[p].wait()
      wh[p] = pltpu.async_copy(hbuf.at[p % 2, pl.ds(0, nb)],
                               fhs_o.at[pl.ds(base + off, nb)], swh)
      gt[p].wait()
      wt[p] = pltpu.async_copy(tbuf.at[p % 2, pl.ds(0, nb)],
                               fts_o.at[pl.ds(base + off, nb)], swt)
  wh[len(sizes) - 1].wait()
  wt[len(sizes) - 1].wait()
  wh[len(sizes) - 2].wait()
  wt[len(sizes) - 2].wait()


def _escore_body(fhs_ref, fts_ref, amat_ref, e_ref):
  s = fhs_ref[...] + fts_ref[...]
  l = jnp.maximum(s, SLOPE * s)
  res = jnp.dot(l, amat_ref[...], preferred_element_type=jnp.float32)
  e_ref[...] = res.T


def _k_escore(fhs, fts, amat):
  be = 2560
  return pl.pallas_call(
      _escore_body,
      grid=(E // be,),
      in_specs=[pl.BlockSpec((be, D), lambda i: (i, 0)),
                pl.BlockSpec((be, D), lambda i: (i, 0)),
                pl.BlockSpec((D, H), lambda i: (0, 0))],
      out_specs=pl.BlockSpec((H, be), lambda i: (0, i)),
      out_shape=jax.ShapeDtypeStruct((H, E), jnp.float32),
  )(fhs, fts, amat)


@functools.partial(
    pl.kernel, mesh=_MESH, compiler_params=_SC_PARAMS,
    out_type=(jax.ShapeDtypeStruct((H, NT, CAPS), jnp.int32),
              jax.ShapeDtypeStruct((H, NT, CAPS), jnp.int32),
              jax.ShapeDtypeStruct((H, NT, CAPS), jnp.float32),
              jax.ShapeDtypeStruct((NT, H), jnp.int32)),
    scratch_types=[pltpu.VMEM((1040,), jnp.int32),
                   pltpu.VMEM((CAPB,), jnp.int32),
                   pltpu.VMEM((CAPM + 16,), jnp.int32),
                   pltpu.VMEM((CAPM + 16,), jnp.int32),
                   pltpu.VMEM((CAPM + 16,), jnp.int32),
                   pltpu.VMEM((CAPM + 16,), jnp.int32),
                   pltpu.VMEM((CAPM + 16,), jnp.float32),
                   pltpu.VMEM((6, 320), jnp.float32),
                   pltpu.VMEM((320,), jnp.float32),
                   pltpu.VMEM((320,), jnp.float32),
                   pltpu.VMEM((320,), jnp.float32),
                   pltpu.VMEM((CAPS + 16,), jnp.int32),
                   pltpu.VMEM((CAPS + 16,), jnp.int32),
                   pltpu.VMEM((CAPS + 16,), jnp.float32),
                   pltpu.VMEM((16,), jnp.int32),
                   pltpu.SemaphoreType.DMA],
)
def _k_seg(e_hbm, src_hbm, bins_hbm, cnt_hbm,
           ssrc_o, sdl_o, sw_o, scnt_o,
           cnt_v, bin_v, eid_v, dl_v, src_v, idx_v, ecol_v,
           tabs_v, deg_v, ctab_v, den_v, stsrc_v, stdl_v, stw_v, c8_v, sem):
  rid = lax.axis_index("c") * 16 + lax.axis_index("s")
  iota = _i16()
  zf16 = jnp.zeros((16,), jnp.float32)
  neg16 = jnp.full((16,), NEG, jnp.float32)

  for t in range(NT):
    pltpu.sync_copy(cnt_hbm.at[t], cnt_v.at[pl.ds(t * NT, NT)])

  def zbody(j, _):
    eid_v[pl.ds(j * 16, 16)] = jnp.zeros((16,), jnp.int32)
    dl_v[pl.ds(j * 16, 16)] = jnp.zeros((16,), jnp.int32)
    idx_v[pl.ds(j * 16, 16)] = jnp.zeros((16,), jnp.int32)
    return 0
  lax.fori_loop(0, (CAPM + 16) // 16, zbody, 0)

  def compact_t(t, m):
    nt = cnt_v[pl.ds(t * 32 + rid, 16)][0]
    pltpu.sync_copy(bins_hbm.at[t, rid], bin_v)

    def cbody(k, m):
      w = bin_v[pl.ds(k * 16, 16)]
      valid = k * 16 + iota < nt
      plsc.store_compressed(eid_v.at[pl.ds(m, 16)], w, mask=valid)
      return m + plsc.all_reduce_population_count(valid)[0]

    return lax.fori_loop(0, (nt + 15) // 16, cbody, m)

  m_tot = 0
  for t in range(NT):
    m_tot = compact_t(t, m_tot)
  m_tot = jnp.minimum(m_tot, CAPM)
  nch = (m_tot + 15) // 16

  for k in range(20):
    deg_v[pl.ds(k * 16, 16)] = zf16

  def degbody(j, _):
    dl = jax.lax.bitwise_and(eid_v[pl.ds(j * 16, 16)], 511)
    valid = j * 16 + iota < m_tot
    _seg_rmw(deg_v, jnp.where(valid, dl, 0),
             jnp.where(valid, 1.0, 0.0), "add")
    return 0
  lax.fori_loop(0, nch, degbody, 0)

  def pbody(j, _):
    dl = jax.lax.bitwise_and(eid_v[pl.ds(j * 16, 16)], 511)
    sk, perm = plsc.sort_key_val(dl, iota)
    pk = jax.lax.shift_left(sk, 4) + perm
    for bi, s in enumerate((1, 2, 4, 8)):
      same = jnp.logical_and(_gath16(sk, jnp.maximum(iota - s, 0)) == sk,
                             iota >= s)
      pk = pk + jax.lax.shift_left(same.astype(jnp.int32), 13 + bi)
    lastm = jnp.logical_or(iota == 15,
                           sk != _gath16(sk, jnp.minimum(iota + 1, 15)))
    pk = pk + jax.lax.shift_left(lastm.astype(jnp.int32), 17)
    dl_v[pl.ds(j * 16, 16)] = pk
    return 0
  lax.fori_loop(0, nch, pbody, 0)

  def logbody(k, _):
    dg = jnp.maximum(deg_v[pl.ds(k * 16, 16)], 1.0)
    bits = plsc.bitcast(dg, jnp.int32)
    ex = jax.lax.shift_right_logical(bits, 23) - 127
    mant = plsc.bitcast(jax.lax.bitwise_or(
        jax.lax.bitwise_and(bits, 0x007FFFFF), 0x3F800000), jnp.float32) - 1.0
    y = (ex.astype(jnp.float32) + mant) * 0.6931472
    for _ in range(3):
      y = y + dg * jnp.exp(-y) - 1.0
    ctab_v[pl.ds(k * 16, 16)] = y * (1.0 / DH)
    return 0
  lax.fori_loop(0, 20, logbody, 0)

  nb_m = (m_tot + 127) // 128

  def uib(j, _):
    idx_v[pl.ds(j * 16, 16)] = jax.lax.shift_right_logical(
        eid_v[pl.ds(j * 16, 16)], 9)
    return 0
  lax.fori_loop(0, nch, uib, 0)

  def srcb(bi, _):
    pltpu.async_copy(src_hbm.at[idx_v.at[pl.ds(bi * 128, 128)]],
                     src_v.at[pl.ds(bi * 128, 128)], sem)
    return 0
  lax.fori_loop(0, nb_m, srcb, 0)

  def srcd(bi, _):
    pltpu.make_async_copy(src_hbm.at[pl.ds(0, 128)],
                          src_v.at[pl.ds(0, 128)], sem).wait()
    return 0
  lax.fori_loop(0, nb_m, srcd, 0)

  c8_v[pl.ds(0, 16)] = jnp.zeros((16,), jnp.int32)

  def head_body(h, _):
    def ib(j, _):
      idx_v[pl.ds(j * 16, 16)] = jax.lax.shift_right_logical(
          eid_v[pl.ds(j * 16, 16)], 9) + h * E
      return 0
    lax.fori_loop(0, nch, ib, 0)

    def eb(bi, _):
      pltpu.async_copy(e_hbm.at[idx_v.at[pl.ds(bi * 128, 128)]],
                       ecol_v.at[pl.ds(bi * 128, 128)], sem)
      return 0
    lax.fori_loop(0, nb_m, eb, 0)

    def ebd(bi, _):
      pltpu.make_async_copy(e_hbm.at[pl.ds(0, 128)],
                            ecol_v.at[pl.ds(0, 128)], sem).wait()
      return 0
    lax.fori_loop(0, nb_m, ebd, 0)
    ecol_v[pl.ds(m_tot, 16)] = neg16

    for r in range(TOPK):
      def tinit(k, _, r=r):
        tabs_v[r, pl.ds(k * 16, 16)] = neg16
        return 0
      lax.fori_loop(0, 20, tinit, 0)

      def rbody(j, _, r=r):
        pk = dl_v[pl.ds(j * 16, 16)]
        b = ecol_v[pl.ds(j * 16, 16)]
        bp = _gath16(b, jax.lax.bitwise_and(pk, 15))
        if r == 0:
          val = bp
        else:
          sk = jax.lax.bitwise_and(jax.lax.shift_right_logical(pk, 4), 511)
          prev = plsc.load_gather(tabs_v.at[r - 1], [sk])
          val = jnp.where(bp < prev, bp, NEG)
        _seg_rmw_pre(tabs_v.at[r], pk, val, "max")
        return 0
      lax.fori_loop(0, nch, rbody, 0)

    def dinit(k, _):
      den_v[pl.ds(k * 16, 16)] = zf16
      return 0
    lax.fori_loop(0, 20, dinit, 0)

    def dbody(j, _):
      pk = dl_v[pl.ds(j * 16, 16)]
      b = ecol_v[pl.ds(j * 16, 16)]
      bp = _gath16(b, jax.lax.bitwise_and(pk, 15))
      sk = jax.lax.bitwise_and(jax.lax.shift_right_logical(pk, 4), 511)
      thr = plsc.load_gather(tabs_v.at[TOPK - 1], [sk])
      mx = plsc.load_gather(tabs_v.at[0], [sk])
      cc = plsc.load_gather(ctab_v, [sk])
      sel = jnp.logical_and(bp >= thr, bp > -1.0e38)
      v = jnp.where(sel, jnp.exp(cc * (bp - mx)), 0.0)
      _seg_rmw_pre(den_v, pk, v, "add")
      return 0
    lax.fori_loop(0, nch, dbody, 0)

    def ebody(j, cnt):
      pk = dl_v[pl.ds(j * 16, 16)]
      b = ecol_v[pl.ds(j * 16, 16)]
      perm = jax.lax.bitwise_and(pk, 15)
      bp = _gath16(b, perm)
      sk = jax.lax.bitwise_and(jax.lax.shift_right_logical(pk, 4), 511)
      thr = plsc.load_gather(tabs_v.at[TOPK - 1], [sk])
      mx = plsc.load_gather(tabs_v.at[0], [sk])
      cc = plsc.load_gather(ctab_v, [sk])
      dn = plsc.load_gather(den_v, [sk])
      sel = jnp.logical_and(bp >= thr, bp > -1.0e38)
      w = jnp.exp(cc * (bp - mx)) / jnp.maximum(dn, 1e-38)
      sv = _gath16(src_v[pl.ds(j * 16, 16)], perm)
      cnt = jnp.minimum(cnt, CAPS)
      plsc.store_compressed(stsrc_v.at[pl.ds(cnt, 16)], sv, mask=sel)
      plsc.store_compressed(stdl_v.at[pl.ds(cnt, 16)], sk, mask=sel)
      plsc.store_compressed(stw_v.at[pl.ds(cnt, 16)], w, mask=sel)
      return cnt + plsc.all_reduce_population_count(sel)[0]
    cnt = lax.fori_loop(0, nch, ebody, 0)
    cnt = jnp.minimum(cnt, CAPS)

    pltpu.sync_copy(stsrc_v.at[pl.ds(0, CAPS)], ssrc_o.at[h, rid])
    pltpu.sync_copy(stdl_v.at[pl.ds(0, CAPS)], sdl_o.at[h, rid])
    pltpu.sync_copy(stw_v.at[pl.ds(0, CAPS)], sw_o.at[h, rid])
    c8_v[pl.ds(0, 16)] = jnp.where(iota == h, cnt, c8_v[pl.ds(0, 16)])
    return 0

  lax.fori_loop(0, H, head_body, 0)
  pltpu.sync_copy(c8_v.at[pl.ds(0, H)], scnt_o.at[rid])


def _hop_body(f_hbm, f0_hbm, ssrc_hbm, sdl_hbm, sw_hbm, scnt_hbm, out_o,
              agg_v, idx_v, gbuf, srcb_v, dlb_v, wb_v, c8_v,
              ab_v, fb_v, sem, last):
  rid = lax.axis_index("c") * 16 + lax.axis_index("s")
  lo = rid * RNG
  iota = _i16()

  def zb(j, _):
    agg_v[pl.ds(j * 16, 16)] = jnp.zeros((16,), jnp.float32)
    return 0
  lax.fori_loop(0, (H * RNG * DH) // 16, zb, 0)

  pltpu.sync_copy(scnt_hbm.at[rid], c8_v.at[pl.ds(0, H)])
  call = c8_v[...]

  for h in range(H):
    nsel = call[h]
    nbb = (nsel + 127) // 128
    pltpu.sync_copy(ssrc_hbm.at[h, rid], srcb_v)
    pltpu.sync_copy(sdl_hbm.at[h, rid], dlb_v)
    pltpu.sync_copy(sw_hbm.at[h, rid], wb_v)

    def ibody(sub, _, h=h):
      o = sub * 16
      valid = o + iota < nsel
      sv = jnp.where(valid, srcb_v[pl.ds(o, 16)], 0)
      idx_v[pl.ds(o, 16)] = sv + h * N
      return 0
    lax.fori_loop(0, nbb * 8, ibody, 0)

    def gfire(bi, _):
      pltpu.async_copy(f_hbm.at[idx_v.at[pl.ds(bi * 128, 128)]],
                       gbuf.at[pl.ds(bi * 128, 128)], sem)
      return 0
    lax.fori_loop(0, nbb, gfire, 0)

    def gdrain(bi, _):
      pltpu.make_async_copy(f_hbm.at[pl.ds(0, 128)],
                            gbuf.at[pl.ds(0, 128)], sem).wait()
      return 0
    lax.fori_loop(0, nbb, gdrain, 0)

    def ssub(sub, _, h=h):
      o = sub * 16
      valid = o + iota < nsel
      wv = jnp.where(valid, wb_v[pl.ds(o, 16)], 0.0)
      dv = jnp.where(valid, dlb_v[pl.ds(o, 16)], 0)
      base16 = dv * DH + h * (RNG * DH)
      for j in range(16):
        row = gbuf[o + j, :] * wv[j]
        b = base16[j]
        agg_v[pl.ds(b, 16)] = agg_v[pl.ds(b, 16)] + row
      return 0
    lax.fori_loop(0, nbb * 8, ssub, 0)

  for h in range(H):
    pltpu.sync_copy(f0_hbm.at[pl.ds(h * N + lo, RNG)], fb_v.at[pl.ds(0, RNG)])

    def blend(j, _, h=h):
      a = agg_v[pl.ds(h * (RNG * DH) + j * DH, 16)]
      ab_v[j, :] = (1.0 - ALPHA) * a + ALPHA * fb_v[j, :]
      return 0
    lax.fori_loop(0, RNG, blend, 0)
    if last:
      pltpu.sync_copy(ab_v.at[pl.ds(0, RNG)],
                      out_o.at[pl.ds(lo, RNG), pl.ds(h * DH, DH)])
    else:
      pltpu.sync_copy(ab_v.at[pl.ds(0, RNG)],
                      out_o.at[pl.ds(h * N + lo, RNG)])


def _make_hop(last):
  out_ty = (jax.ShapeDtypeStruct((N, D), jnp.float32) if last
            else jax.ShapeDtypeStruct((H * N, DH), jnp.float32))
  return functools.partial(
      pl.kernel, mesh=_MESH, compiler_params=_SC_PARAMS,
      out_type=out_ty,
      scratch_types=[pltpu.VMEM((H * RNG * DH,), jnp.float32),
                     pltpu.VMEM((CAPS,), jnp.int32),
                     pltpu.VMEM((CAPS, DH), jnp.float32),
                     pltpu.VMEM((CAPS,), jnp.int32),
                     pltpu.VMEM((CAPS,), jnp.int32),
                     pltpu.VMEM((CAPS,), jnp.float32),
                     pltpu.VMEM((16,), jnp.int32),
                     pltpu.VMEM((320, DH), jnp.float32),
                     pltpu.VMEM((320, DH), jnp.float32),
                     pltpu.SemaphoreType.DMA],
  )(functools.partial(_hop_body, last=last))


_k_hop_mid = _make_hop(False)
_k_hop_last = _make_hop(True)


def _ffn_body(f_ref, feat_ref, g_ref, b_ref, w1_ref, b1_ref, w2_ref, b2_ref,
              out_ref):
  rst = f_ref[...] + feat_ref[...]
  mu = jnp.mean(rst, axis=-1, keepdims=True)
  var = jnp.mean(jnp.square(rst - mu), axis=-1, keepdims=True)
  y = (rst - mu) * jax.lax.rsqrt(var + 1e-5) * g_ref[...] + b_ref[...]
  hdn = jnp.maximum(
      jnp.dot(y, w1_ref[...], preferred_element_type=jnp.float32)
      + b1_ref[...], 0.0)
  out_ref[...] = (jnp.dot(hdn, w2_ref[...], preferred_element_type=jnp.float32)
                  + b2_ref[...] + rst)


def _k_ffn(f2d, feat, g, b, w1, b1, w2, b2):
  bn = 400
  return pl.pallas_call(
      _ffn_body,
      grid=(N // bn,),
      in_specs=[pl.BlockSpec((bn, D), lambda i: (i, 0)),
                pl.BlockSpec((bn, D), lambda i: (i, 0)),
                pl.BlockSpec((D,), lambda i: (0,)),
                pl.BlockSpec((D,), lambda i: (0,)),
                pl.BlockSpec((D, 4 * D), lambda i: (0, 0)),
                pl.BlockSpec((4 * D,), lambda i: (0,)),
                pl.BlockSpec((4 * D, D), lambda i: (0, 0)),
                pl.BlockSpec((D,), lambda i: (0,))],
      out_specs=pl.BlockSpec((bn, D), lambda i: (i, 0)),
      out_shape=jax.ShapeDtypeStruct((N, D), jnp.float32),
  )(f2d, feat, g, b, w1, b1, w2, b2)


def kernel(feat, edge_index, W_head, W_tail, W_ent, attn,
           ln1_g, ln1_b, ln2_g, ln2_b, W_ff1, b_ff1, W_ff2, b_ff2):
  src = edge_index[0].astype(jnp.int32)
  dst = edge_index[1].astype(jnp.int32)
  attn2 = attn.reshape(H, DH)
  amat = (jnp.eye(H, dtype=jnp.float32)[:, None, :]
          * attn2[:, :, None]).reshape(D, H)

  fh, ft, f0 = _k_pre(feat, W_head, W_tail, W_ent, ln1_g, ln1_b)
  f0_flat = f0.reshape(H * N, DH)

  bins, bcnt = _k_bin(dst)
  fhs, fts = _k_gath(fh, ft, src, dst)
  e = _k_escore(fhs, fts, amat).reshape(H * E)

  ssrc, sdl, sw, scnt = _k_seg(e, src, bins, bcnt)

  f = f0_flat
  for _ in range(HOP - 1):
    f = _k_hop_mid(f, f0_flat, ssrc, sdl, sw, scnt)
  f2d = _k_hop_last(f, f0_flat, ssrc, sdl, sw, scnt)

  return _k_ffn(f2d, feat, ln2_g, ln2_b, W_ff1, b_ff1, W_ff2, b_ff2)


# speedup vs baseline: 84.1656x; 1.0373x over previous
import functools

import jax
import jax.numpy as jnp
from jax import lax
from jax.experimental import pallas as pl
from jax.experimental.pallas import tpu as pltpu
from jax.experimental.pallas import tpu_sc as plsc

N = 10000
E = 320000
D = 128
H = 8
DH = 16
HOP = 5
ALPHA = 0.1
TOPK = 5
SLOPE = 0.2

NT = 32
RNG = 313
ESH = E // NT
CAPB = 512
CAPM = 16384
CAPS = 2048
NEG = -3.0e38
BIG = 3.0e38

_SC_PARAMS = pltpu.CompilerParams(needs_layout_passes=False,
                                  use_tc_tiling_on_sc=False)
_MESH = plsc.VectorSubcoreMesh(core_axis_name="c", subcore_axis_name="s")
_GDN = jax.lax.GatherDimensionNumbers((), (0,), (0,))
_IN_BOUNDS = jax.lax.GatherScatterMode.PROMISE_IN_BOUNDS


def _i16():
  return lax.iota(jnp.int32, 16)


def _gath16(v, idx):
  return jax.lax.gather(v, idx[:, None], _GDN, (1,), mode=_IN_BOUNDS)


def _prefix_combine(pk, pv, op):
  iota = _i16()
  for bi, s in enumerate((1, 2, 4, 8)):
    same = jax.lax.bitwise_and(
        jax.lax.shift_right_logical(pk, 13 + bi), 1) == 1
    shifted = _gath16(pv, jnp.maximum(iota - s, 0))
    if op == "max":
      pv = jnp.where(same, jnp.maximum(pv, shifted), pv)
    else:
      pv = pv + jnp.where(same, shifted, jnp.zeros_like(pv))
  return pv


def _seg_rmw_pre(tab, pk, pv, op):
  sk = jax.lax.bitwise_and(jax.lax.shift_right_logical(pk, 4), 511)
  pv = _prefix_combine(pk, pv, op)
  lastm = jax.lax.bitwise_and(jax.lax.shift_right_logical(pk, 17), 1) == 1
  cur = plsc.load_gather(tab, [sk])
  nv = jnp.maximum(cur, pv) if op == "max" else cur + pv
  plsc.store_scatter(tab, [sk], nv, mask=lastm)


def _seg_rmw(tab, dl, val, op):
  iota = _i16()
  sk, sv = plsc.sort_key_val(dl, iota)
  pv = _gath16(val, sv)
  for s in (1, 2, 4, 8):
    src_lane = jnp.maximum(iota - s, 0)
    same = jnp.logical_and(_gath16(sk, src_lane) == sk, iota >= s)
    shifted = _gath16(pv, src_lane)
    if op == "max":
      pv = jnp.where(same, jnp.maximum(pv, shifted), pv)
    else:
      pv = pv + jnp.where(same, shifted, jnp.zeros_like(pv))
  nxt = _gath16(sk, jnp.minimum(iota + 1, 15))
  lastm = jnp.logical_or(iota == 15, sk != nxt)
  cur = plsc.load_gather(tab, [sk])
  nv = jnp.maximum(cur, pv) if op == "max" else cur + pv
  plsc.store_scatter(tab, [sk], nv, mask=lastm)


def _pre_body(feat_ref, wh_ref, wt_ref, we_ref, g_ref, b_ref,
              fh_ref, ft_ref, f0_ref):
  x = feat_ref[...]
  mu = jnp.mean(x, axis=-1, keepdims=True)
  var = jnp.mean(jnp.square(x - mu), axis=-1, keepdims=True)
  xn = (x - mu) * jax.lax.rsqrt(var + 1e-5) * g_ref[...] + b_ref[...]
  fh_ref[...] = jnp.dot(xn, wh_ref[...], preferred_element_type=jnp.float32)
  ft_ref[...] = jnp.dot(xn, wt_ref[...], preferred_element_type=jnp.float32)
  fe = jnp.dot(xn, we_ref[...], preferred_element_type=jnp.float32)
  for h in range(H):
    f0_ref[h] = fe[:, h * DH:(h + 1) * DH]


def _k_pre(feat, wh, wt, we, g, b):
  bn = 400
  return pl.pallas_call(
      _pre_body,
      grid=(N // bn,),
      in_specs=[pl.BlockSpec((bn, D), lambda i: (i, 0)),
                pl.BlockSpec((D, D), lambda i: (0, 0)),
                pl.BlockSpec((D, D), lambda i: (0, 0)),
                pl.BlockSpec((D, D), lambda i: (0, 0)),
                pl.BlockSpec((D,), lambda i: (0,)),
                pl.BlockSpec((D,), lambda i: (0,))],
      out_specs=[pl.BlockSpec((bn, D), lambda i: (i, 0)),
                 pl.BlockSpec((bn, D), lambda i: (i, 0)),
                 pl.BlockSpec((H, bn, DH), lambda i: (0, i, 0))],
      out_shape=[jax.ShapeDtypeStruct((N, D), jnp.float32),
                 jax.ShapeDtypeStruct((N, D), jnp.float32),
                 jax.ShapeDtypeStruct((H, N, DH), jnp.float32)],
  )(feat, wh, wt, we, g, b)


@functools.partial(
    pl.kernel, mesh=_MESH, compiler_params=_SC_PARAMS,
    out_type=(jax.ShapeDtypeStruct((NT, NT, CAPB), jnp.int32),
              jax.ShapeDtypeStruct((NT, NT), jnp.int32)),
    scratch_types=[pltpu.VMEM((ESH,), jnp.int32),
                   pltpu.VMEM((NT, CAPB), jnp.int32),
                   pltpu.VMEM((48,), jnp.int32)],
)
def _k_bin(dst_hbm, bins_o, cnt_o, shard_v, bins_v, cnt_v):
  tid = lax.axis_index("c") * 16 + lax.axis_index("s")
  pltpu.sync_copy(dst_hbm.at[pl.ds(tid * ESH, ESH)], shard_v)
  for k in range(2):
    cnt_v[pl.ds(16 * k, 16)] = jnp.zeros((16,), jnp.int32)
  iota = _i16()

  def body(j, _):
    d = shard_v[pl.ds(j * 16, 16)]
    r = d // RNG
    dl = d - r * RNG
    eid = tid * ESH + j * 16 + iota
    word = eid * 512 + dl
    sk, sv = plsc.sort_key_val(r, word)
    bnd = jnp.logical_or(iota == 0, _gath16(sk, jnp.maximum(iota - 1, 0)) != sk)
    first = plsc.cummax(jnp.where(bnd, iota, -1))
    rank = iota - first
    base = plsc.load_gather(cnt_v, [sk])
    pos = jnp.minimum(base + rank, CAPB - 1)
    plsc.store_scatter(bins_v, [sk, pos], sv)
    lastm = jnp.logical_or(iota == 15, _gath16(sk, jnp.minimum(iota + 1, 15)) != sk)
    plsc.store_scatter(cnt_v, [sk], jnp.minimum(base + rank + 1, CAPB), mask=lastm)
    return 0

  lax.fori_loop(0, ESH // 16, body, 0)
  pltpu.sync_copy(bins_v, bins_o.at[tid])
  pltpu.sync_copy(cnt_v.at[pl.ds(0, NT)], cnt_o.at[tid])


@functools.partial(
    pl.kernel, mesh=_MESH, compiler_params=_SC_PARAMS,
    out_type=(jax.ShapeDtypeStruct((E, D), jnp.float32),
              jax.ShapeDtypeStruct((E, D), jnp.float32)),
    scratch_types=[pltpu.VMEM((ESH,), jnp.int32),
                   pltpu.VMEM((ESH,), jnp.int32),
                   pltpu.VMEM((2, 128, D), jnp.float32),
                   pltpu.VMEM((2, 128, D), jnp.float32),
                   pltpu.SemaphoreType.DMA,
                   pltpu.SemaphoreType.DMA,
                   pltpu.SemaphoreType.DMA,
                   pltpu.SemaphoreType.DMA],
)
def _k_gath(fh_hbm, ft_hbm, src_hbm, dst_hbm, fhs_o, fts_o,
            src_v, dst_v, hbuf, tbuf, sgh, swh, sgt, swt):
  tid = lax.axis_index("c") * 16 + lax.axis_index("s")
  base = tid * ESH
  pltpu.sync_copy(src_hbm.at[pl.ds(base, ESH)], src_v)
  pltpu.sync_copy(dst_hbm.at[pl.ds(base, ESH)], dst_v)

  nb_full = ESH // 128
  sizes = [128] * nb_full + [16]
  gh = {}
  wh = {}
  gt = {}
  wt = {}
  for b in range(len(sizes) + 1):
    if b < len(sizes):
      if b >= 2:
        wh[b - 2].wait()
        wt[b - 2].wait()
      off = b * 128
      nb = sizes[b]
      gh[b] = pltpu.async_copy(fh_hbm.at[src_v.at[pl.ds(off, nb)]],
                               hbuf.at[b % 2, pl.ds(0, nb)], sgh)
      gt[b] = pltpu.async_copy(ft_hbm.at[dst_v.at[pl.ds(off, nb)]],
                               tbuf.at[b % 2, pl.ds(0, nb)], sgt)
    if b >= 1:
      p = b - 1
      off = p * 128
      nb = sizes[p]
      gh[p].wait()
      wh[p] = pltpu.async_copy(hbuf.at[p % 2, pl.ds(0, nb)],
                               fhs_o.at[pl.ds(base + off, nb)], swh)
      gt[p].wait()
      wt[p] = pltpu.async_copy(tbuf.at[p % 2, pl.ds(0, nb)],
                               fts_o.at[pl.ds(base + off, nb)], swt)
  wh[len(sizes) - 1].wait()
  wt[len(sizes) - 1].wait()
  wh[len(sizes) - 2].wait()
  wt[len(sizes) - 2].wait()


def _escore_body(fhs_ref, fts_ref, amat_ref, e_ref):
  s = fhs_ref[...] + fts_ref[...]
  l = jnp.maximum(s, SLOPE * s)
  res = jnp.dot(l, amat_ref[...], preferred_element_type=jnp.float32)
  e_ref[...] = res.T


def _k_escore(fhs, fts, amat):
  be = 2560
  return pl.pallas_call(
      _escore_body,
      grid=(E // be,),
      in_specs=[pl.BlockSpec((be, D), lambda i: (i, 0)),
                pl.BlockSpec((be, D), lambda i: (i, 0)),
                pl.BlockSpec((D, H), lambda i: (0, 0))],
      out_specs=pl.BlockSpec((H, be), lambda i: (0, i)),
      out_shape=jax.ShapeDtypeStruct((H, E), jnp.float32),
  )(fhs, fts, amat)


@functools.partial(
    pl.kernel, mesh=_MESH, compiler_params=_SC_PARAMS,
    out_type=(jax.ShapeDtypeStruct((H, NT, CAPS), jnp.int32),
              jax.ShapeDtypeStruct((H, NT, CAPS), jnp.int32),
              jax.ShapeDtypeStruct((H, NT, CAPS), jnp.float32),
              jax.ShapeDtypeStruct((NT, H), jnp.int32)),
    scratch_types=[pltpu.VMEM((1040,), jnp.int32),
                   pltpu.VMEM((CAPB,), jnp.int32),
                   pltpu.VMEM((CAPM + 16,), jnp.int32),
                   pltpu.VMEM((CAPM + 16,), jnp.int32),
                   pltpu.VMEM((CAPM + 16,), jnp.int32),
                   pltpu.VMEM((CAPM + 16,), jnp.int32),
                   pltpu.VMEM((CAPM + 16,), jnp.float32),
                   pltpu.VMEM((6, 320), jnp.float32),
                   pltpu.VMEM((320,), jnp.float32),
                   pltpu.VMEM((320,), jnp.float32),
                   pltpu.VMEM((320,), jnp.float32),
                   pltpu.VMEM((CAPS + 16,), jnp.int32),
                   pltpu.VMEM((CAPS + 16,), jnp.int32),
                   pltpu.VMEM((CAPS + 16,), jnp.float32),
                   pltpu.VMEM((16,), jnp.int32),
                   pltpu.SemaphoreType.DMA],
)
def _k_seg(e_hbm, src_hbm, bins_hbm, cnt_hbm,
           ssrc_o, sdl_o, sw_o, scnt_o,
           cnt_v, bin_v, eid_v, dl_v, src_v, idx_v, ecol_v,
           tabs_v, deg_v, ctab_v, den_v, stsrc_v, stdl_v, stw_v, c8_v, sem):
  rid = lax.axis_index("c") * 16 + lax.axis_index("s")
  iota = _i16()
  zf16 = jnp.zeros((16,), jnp.float32)
  neg16 = jnp.full((16,), NEG, jnp.float32)

  for t in range(NT):
    pltpu.sync_copy(cnt_hbm.at[t], cnt_v.at[pl.ds(t * NT, NT)])

  def zbody(j, _):
    eid_v[pl.ds(j * 16, 16)] = jnp.zeros((16,), jnp.int32)
    dl_v[pl.ds(j * 16, 16)] = jnp.zeros((16,), jnp.int32)
    idx_v[pl.ds(j * 16, 16)] = jnp.zeros((16,), jnp.int32)
    return 0
  lax.fori_loop(0, (CAPM + 16) // 16, zbody, 0)

  def compact_t(t, m):
    nt = cnt_v[pl.ds(t * 32 + rid, 16)][0]
    pltpu.sync_copy(bins_hbm.at[t, rid], bin_v)

    def cbody(k, m):
      w = bin_v[pl.ds(k * 16, 16)]
      valid = k * 16 + iota < nt
      plsc.store_compressed(eid_v.at[pl.ds(m, 16)], w, mask=valid)
      return m + plsc.all_reduce_population_count(valid)[0]

    return lax.fori_loop(0, (nt + 15) // 16, cbody, m)

  m_tot = 0
  for t in range(NT):
    m_tot = compact_t(t, m_tot)
  m_tot = jnp.minimum(m_tot, CAPM)
  nch = (m_tot + 15) // 16

  for k in range(20):
    deg_v[pl.ds(k * 16, 16)] = zf16

  def degbody(j, _):
    dl = jax.lax.bitwise_and(eid_v[pl.ds(j * 16, 16)], 511)
    valid = j * 16 + iota < m_tot
    _seg_rmw(deg_v, jnp.where(valid, dl, 0),
             jnp.where(valid, 1.0, 0.0), "add")
    return 0
  lax.fori_loop(0, nch, degbody, 0)

  def pbody(j, _):
    dl = jax.lax.bitwise_and(eid_v[pl.ds(j * 16, 16)], 511)
    sk, perm = plsc.sort_key_val(dl, iota)
    pk = jax.lax.shift_left(sk, 4) + perm
    for bi, s in enumerate((1, 2, 4, 8)):
      same = jnp.logical_and(_gath16(sk, jnp.maximum(iota - s, 0)) == sk,
                             iota >= s)
      pk = pk + jax.lax.shift_left(same.astype(jnp.int32), 13 + bi)
    lastm = jnp.logical_or(iota == 15,
                           sk != _gath16(sk, jnp.minimum(iota + 1, 15)))
    pk = pk + jax.lax.shift_left(lastm.astype(jnp.int32), 17)
    dl_v[pl.ds(j * 16, 16)] = pk
    return 0
  lax.fori_loop(0, nch, pbody, 0)

  def logbody(k, _):
    dg = jnp.maximum(deg_v[pl.ds(k * 16, 16)], 1.0)
    bits = plsc.bitcast(dg, jnp.int32)
    ex = jax.lax.shift_right_logical(bits, 23) - 127
    mant = plsc.bitcast(jax.lax.bitwise_or(
        jax.lax.bitwise_and(bits, 0x007FFFFF), 0x3F800000), jnp.float32) - 1.0
    y = (ex.astype(jnp.float32) + mant) * 0.6931472
    for _ in range(3):
      y = y + dg * jnp.exp(-y) - 1.0
    ctab_v[pl.ds(k * 16, 16)] = y * (1.0 / DH)
    return 0
  lax.fori_loop(0, 20, logbody, 0)

  nb_m = (m_tot + 127) // 128

  def uib(j, _):
    idx_v[pl.ds(j * 16, 16)] = jax.lax.shift_right_logical(
        eid_v[pl.ds(j * 16, 16)], 9)
    return 0
  lax.fori_loop(0, nch, uib, 0)

  def srcb(bi, _):
    pltpu.async_copy(src_hbm.at[idx_v.at[pl.ds(bi * 128, 128)]],
                     src_v.at[pl.ds(bi * 128, 128)], sem)
    return 0
  lax.fori_loop(0, nb_m, srcb, 0)

  def srcd(bi, _):
    pltpu.make_async_copy(src_hbm.at[pl.ds(0, 128)],
                          src_v.at[pl.ds(0, 128)], sem).wait()
    return 0
  lax.fori_loop(0, nb_m, srcd, 0)

  c8_v[pl.ds(0, 16)] = jnp.zeros((16,), jnp.int32)

  def head_body(h, _):
    def ib(j, _):
      idx_v[pl.ds(j * 16, 16)] = jax.lax.shift_right_logical(
          eid_v[pl.ds(j * 16, 16)], 9) + h * E
      return 0
    lax.fori_loop(0, nch, ib, 0)

    def eb(bi, _):
      pltpu.async_copy(e_hbm.at[idx_v.at[pl.ds(bi * 128, 128)]],
                       ecol_v.at[pl.ds(bi * 128, 128)], sem)
      return 0
    lax.fori_loop(0, nb_m, eb, 0)

    def ebd(bi, _):
      pltpu.make_async_copy(e_hbm.at[pl.ds(0, 128)],
                            ecol_v.at[pl.ds(0, 128)], sem).wait()
      return 0
    lax.fori_loop(0, nb_m, ebd, 0)
    ecol_v[pl.ds(m_tot, 16)] = neg16

    for r in range(TOPK):
      def tinit(k, _, r=r):
        tabs_v[r, pl.ds(k * 16, 16)] = neg16
        return 0
      lax.fori_loop(0, 20, tinit, 0)

      def rbody(j, _, r=r):
        pk = dl_v[pl.ds(j * 16, 16)]
        b = ecol_v[pl.ds(j * 16, 16)]
        bp = _gath16(b, jax.lax.bitwise_and(pk, 15))
        if r == 0:
          val = bp
        else:
          sk = jax.lax.bitwise_and(jax.lax.shift_right_logical(pk, 4), 511)
          prev = plsc.load_gather(tabs_v.at[r - 1], [sk])
          val = jnp.where(bp < prev, bp, NEG)
        _seg_rmw_pre(tabs_v.at[r], pk, val, "max")
        return 0
      lax.fori_loop(0, nch, rbody, 0)

    def dinit(k, _):
      den_v[pl.ds(k * 16, 16)] = zf16
      return 0
    lax.fori_loop(0, 20, dinit, 0)

    def dbody(j, _):
      pk = dl_v[pl.ds(j * 16, 16)]
      b = ecol_v[pl.ds(j * 16, 16)]
      bp = _gath16(b, jax.lax.bitwise_and(pk, 15))
      sk = jax.lax.bitwise_and(jax.lax.shift_right_logical(pk, 4), 511)
      thr = plsc.load_gather(tabs_v.at[TOPK - 1], [sk])
      mx = plsc.load_gather(tabs_v.at[0], [sk])
      cc = plsc.load_gather(ctab_v, [sk])
      sel = jnp.logical_and(bp >= thr, bp > -1.0e38)
      v = jnp.where(sel, jnp.exp(cc * (bp - mx)), 0.0)
      _seg_rmw_pre(den_v, pk, v, "add")
      return 0
    lax.fori_loop(0, nch, dbody, 0)

    def ebody(j, cnt):
      pk = dl_v[pl.ds(j * 16, 16)]
      b = ecol_v[pl.ds(j * 16, 16)]
      perm = jax.lax.bitwise_and(pk, 15)
      bp = _gath16(b, perm)
      sk = jax.lax.bitwise_and(jax.lax.shift_right_logical(pk, 4), 511)
      thr = plsc.load_gather(tabs_v.at[TOPK - 1], [sk])
      mx = plsc.load_gather(tabs_v.at[0], [sk])
      cc = plsc.load_gather(ctab_v, [sk])
      dn = plsc.load_gather(den_v, [sk])
      sel = jnp.logical_and(bp >= thr, bp > -1.0e38)
      w = jnp.exp(cc * (bp - mx)) / jnp.maximum(dn, 1e-38)
      sv = _gath16(src_v[pl.ds(j * 16, 16)], perm)
      cnt = jnp.minimum(cnt, CAPS)
      plsc.store_compressed(stsrc_v.at[pl.ds(cnt, 16)], sv, mask=sel)
      plsc.store_compressed(stdl_v.at[pl.ds(cnt, 16)], sk, mask=sel)
      plsc.store_compressed(stw_v.at[pl.ds(cnt, 16)], w, mask=sel)
      return cnt + plsc.all_reduce_population_count(sel)[0]
    cnt = lax.fori_loop(0, nch, ebody, 0)
    cnt = jnp.minimum(cnt, CAPS)

    pltpu.sync_copy(stsrc_v.at[pl.ds(0, CAPS)], ssrc_o.at[h, rid])
    pltpu.sync_copy(stdl_v.at[pl.ds(0, CAPS)], sdl_o.at[h, rid])
    pltpu.sync_copy(stw_v.at[pl.ds(0, CAPS)], sw_o.at[h, rid])
    c8_v[pl.ds(0, 16)] = jnp.where(iota == h, cnt, c8_v[pl.ds(0, 16)])
    return 0

  lax.fori_loop(0, H, head_body, 0)
  pltpu.sync_copy(c8_v.at[pl.ds(0, H)], scnt_o.at[rid])


def _hop_body(f_hbm, f0_hbm, ssrc_hbm, sdl_hbm, sw_hbm, scnt_hbm, out_o,
              agg_v, idx_v, gbuf, srcb_v, dlb_v, wb_v, c8_v,
              ab_v, fb_v, sem, last):
  rid = lax.axis_index("c") * 16 + lax.axis_index("s")
  lo = rid * RNG
  iota = _i16()

  z16 = jnp.zeros((16,), jnp.float32)

  def zb(j, _):
    for u in range(8):
      agg_v[pl.ds(j * 128 + u * 16, 16)] = z16
    return 0
  lax.fori_loop(0, (H * RNG * DH) // 128, zb, 0)
  def zt(j, _):
    agg_v[pl.ds((H * RNG * DH) // 128 * 128 + j * 16, 16)] = z16
    return 0
  lax.fori_loop(0, (H * RNG * DH) % 128 // 16, zt, 0)

  pltpu.sync_copy(scnt_hbm.at[rid], c8_v.at[pl.ds(0, H)])
  call = c8_v[...]

  for h in range(H):
    nsel = call[h]
    nbb = (nsel + 127) // 128
    d1 = pltpu.async_copy(ssrc_hbm.at[h, rid], srcb_v, sem)
    d2 = pltpu.async_copy(sdl_hbm.at[h, rid], dlb_v, sem)
    d3 = pltpu.async_copy(sw_hbm.at[h, rid], wb_v, sem)
    d1.wait(); d2.wait(); d3.wait()

    def ibody(sub, _, h=h):
      o = sub * 16
      valid = o + iota < nsel
      sv = jnp.where(valid, srcb_v[pl.ds(o, 16)], 0)
      idx_v[pl.ds(o, 16)] = sv + h * N
      return 0
    lax.fori_loop(0, nbb * 8, ibody, 0)

    def gfire(bi, _):
      pltpu.async_copy(f_hbm.at[idx_v.at[pl.ds(bi * 128, 128)]],
                       gbuf.at[pl.ds(bi * 128, 128)], sem)
      return 0
    lax.fori_loop(0, nbb, gfire, 0)

    def gdrain(bi, _):
      pltpu.make_async_copy(f_hbm.at[pl.ds(0, 128)],
                            gbuf.at[pl.ds(0, 128)], sem).wait()
      return 0
    lax.fori_loop(0, nbb, gdrain, 0)

    def ssub(sub, _, h=h):
      o = sub * 16
      valid = o + iota < nsel
      wv = jnp.where(valid, wb_v[pl.ds(o, 16)], 0.0)
      dv = jnp.where(valid, dlb_v[pl.ds(o, 16)], 0)
      base16 = dv * DH + h * (RNG * DH)
      for j in range(16):
        row = gbuf[o + j, :] * wv[j]
        b = base16[j]
        agg_v[pl.ds(b, 16)] = agg_v[pl.ds(b, 16)] + row
      return 0
    lax.fori_loop(0, nbb * 8, ssub, 0)

  for h in range(H):
    pltpu.sync_copy(f0_hbm.at[pl.ds(h * N + lo, RNG)], fb_v.at[pl.ds(0, RNG)])

    def blend(j, _, h=h):
      a = agg_v[pl.ds(h * (RNG * DH) + j * DH, 16)]
      ab_v[j, :] = (1.0 - ALPHA) * a + ALPHA * fb_v[j, :]
      return 0
    lax.fori_loop(0, RNG, blend, 0)
    if last:
      pltpu.sync_copy(ab_v.at[pl.ds(0, RNG)],
                      out_o.at[pl.ds(lo, RNG), pl.ds(h * DH, DH)])
    else:
      pltpu.sync_copy(ab_v.at[pl.ds(0, RNG)],
                      out_o.at[pl.ds(h * N + lo, RNG)])


def _make_hop(last):
  out_ty = (jax.ShapeDtypeStruct((N, D), jnp.float32) if last
            else jax.ShapeDtypeStruct((H * N, DH), jnp.float32))
  return functools.partial(
      pl.kernel, mesh=_MESH, compiler_params=_SC_PARAMS,
      out_type=out_ty,
      scratch_types=[pltpu.VMEM((H * RNG * DH,), jnp.float32),
                     pltpu.VMEM((CAPS,), jnp.int32),
                     pltpu.VMEM((CAPS, DH), jnp.float32),
                     pltpu.VMEM((CAPS,), jnp.int32),
                     pltpu.VMEM((CAPS,), jnp.int32),
                     pltpu.VMEM((CAPS,), jnp.float32),
                     pltpu.VMEM((16,), jnp.int32),
                     pltpu.VMEM((320, DH), jnp.float32),
                     pltpu.VMEM((320, DH), jnp.float32),
                     pltpu.SemaphoreType.DMA],
  )(functools.partial(_hop_body, last=last))


_k_hop_mid = _make_hop(False)
_k_hop_last = _make_hop(True)


def _ffn_body(f_ref, feat_ref, g_ref, b_ref, w1_ref, b1_ref, w2_ref, b2_ref,
              out_ref):
  rst = f_ref[...] + feat_ref[...]
  mu = jnp.mean(rst, axis=-1, keepdims=True)
  var = jnp.mean(jnp.square(rst - mu), axis=-1, keepdims=True)
  y = (rst - mu) * jax.lax.rsqrt(var + 1e-5) * g_ref[...] + b_ref[...]
  hdn = jnp.maximum(
      jnp.dot(y, w1_ref[...], preferred_element_type=jnp.float32)
      + b1_ref[...], 0.0)
  out_ref[...] = (jnp.dot(hdn, w2_ref[...], preferred_element_type=jnp.float32)
                  + b2_ref[...] + rst)


def _k_ffn(f2d, feat, g, b, w1, b1, w2, b2):
  bn = 400
  return pl.pallas_call(
      _ffn_body,
      grid=(N // bn,),
      in_specs=[pl.BlockSpec((bn, D), lambda i: (i, 0)),
                pl.BlockSpec((bn, D), lambda i: (i, 0)),
                pl.BlockSpec((D,), lambda i: (0,)),
                pl.BlockSpec((D,), lambda i: (0,)),
                pl.BlockSpec((D, 4 * D), lambda i: (0, 0)),
                pl.BlockSpec((4 * D,), lambda i: (0,)),
                pl.BlockSpec((4 * D, D), lambda i: (0, 0)),
                pl.BlockSpec((D,), lambda i: (0,))],
      out_specs=pl.BlockSpec((bn, D), lambda i: (i, 0)),
      out_shape=jax.ShapeDtypeStruct((N, D), jnp.float32),
  )(f2d, feat, g, b, w1, b1, w2, b2)


def kernel(feat, edge_index, W_head, W_tail, W_ent, attn,
           ln1_g, ln1_b, ln2_g, ln2_b, W_ff1, b_ff1, W_ff2, b_ff2):
  src = edge_index[0].astype(jnp.int32)
  dst = edge_index[1].astype(jnp.int32)
  attn2 = attn.reshape(H, DH)
  amat = (jnp.eye(H, dtype=jnp.float32)[:, None, :]
          * attn2[:, :, None]).reshape(D, H)

  fh, ft, f0 = _k_pre(feat, W_head, W_tail, W_ent, ln1_g, ln1_b)
  f0_flat = f0.reshape(H * N, DH)

  bins, bcnt = _k_bin(dst)
  fhs, fts = _k_gath(fh, ft, src, dst)
  e = _k_escore(fhs, fts, amat).reshape(H * E)

  ssrc, sdl, sw, scnt = _k_seg(e, src, bins, bcnt)

  f = f0_flat
  for _ in range(HOP - 1):
    f = _k_hop_mid(f, f0_flat, ssrc, sdl, sw, scnt)
  f2d = _k_hop_last(f, f0_flat, ssrc, sdl, sw, scnt)

  return _k_ffn(f2d, feat, ln2_g, ln2_b, W_ff1, b_ff1, W_ff2, b_ff2)


# speedup vs baseline: 84.1950x; 1.0003x over previous
import functools

import jax
import jax.numpy as jnp
from jax import lax
from jax.experimental import pallas as pl
from jax.experimental.pallas import tpu as pltpu
from jax.experimental.pallas import tpu_sc as plsc

N = 10000
E = 320000
D = 128
H = 8
DH = 16
HOP = 5
ALPHA = 0.1
TOPK = 5
SLOPE = 0.2

NT = 32
RNG = 313
ESH = E // NT
CAPB = 512
CAPM = 16384
CAPS = 2048
NEG = -3.0e38

_SC_PARAMS = pltpu.CompilerParams(needs_layout_passes=False,
                                  use_tc_tiling_on_sc=False)
_MESH = plsc.VectorSubcoreMesh(core_axis_name="c", subcore_axis_name="s")
_GDN = jax.lax.GatherDimensionNumbers((), (0,), (0,))
_IN_BOUNDS = jax.lax.GatherScatterMode.PROMISE_IN_BOUNDS


def _i16():
  return lax.iota(jnp.int32, 16)


def _gath16(v, idx):
  return jax.lax.gather(v, idx[:, None], _GDN, (1,), mode=_IN_BOUNDS)


def _prefix_combine(pk, pv, op):
  iota = _i16()
  for bi, s in enumerate((1, 2, 4, 8)):
    same = jax.lax.bitwise_and(
        jax.lax.shift_right_logical(pk, 13 + bi), 1) == 1
    shifted = _gath16(pv, jnp.maximum(iota - s, 0))
    if op == "max":
      pv = jnp.where(same, jnp.maximum(pv, shifted), pv)
    else:
      pv = pv + jnp.where(same, shifted, jnp.zeros_like(pv))
  return pv


def _seg_rmw_pre(tab, pk, pv, op):
  sk = jax.lax.bitwise_and(jax.lax.shift_right_logical(pk, 4), 511)
  pv = _prefix_combine(pk, pv, op)
  lastm = jax.lax.bitwise_and(jax.lax.shift_right_logical(pk, 17), 1) == 1
  cur = plsc.load_gather(tab, [sk])
  nv = jnp.maximum(cur, pv) if op == "max" else cur + pv
  plsc.store_scatter(tab, [sk], nv, mask=lastm)


def _seg_rmw(tab, dl, val, op):
  iota = _i16()
  sk, sv = plsc.sort_key_val(dl, iota)
  pv = _gath16(val, sv)
  for s in (1, 2, 4, 8):
    src_lane = jnp.maximum(iota - s, 0)
    same = jnp.logical_and(_gath16(sk, src_lane) == sk, iota >= s)
    shifted = _gath16(pv, src_lane)
    if op == "max":
      pv = jnp.where(same, jnp.maximum(pv, shifted), pv)
    else:
      pv = pv + jnp.where(same, shifted, jnp.zeros_like(pv))
  nxt = _gath16(sk, jnp.minimum(iota + 1, 15))
  lastm = jnp.logical_or(iota == 15, sk != nxt)
  cur = plsc.load_gather(tab, [sk])
  nv = jnp.maximum(cur, pv) if op == "max" else cur + pv
  plsc.store_scatter(tab, [sk], nv, mask=lastm)


def _pre_body(feat_ref, wh_ref, wt_ref, we_ref, g_ref, b_ref,
              fh_ref, ft_ref, f0_ref):
  x = feat_ref[...]
  mu = jnp.mean(x, axis=-1, keepdims=True)
  var = jnp.mean(jnp.square(x - mu), axis=-1, keepdims=True)
  xn = (x - mu) * jax.lax.rsqrt(var + 1e-5) * g_ref[...] + b_ref[...]
  fh_ref[...] = jnp.dot(xn, wh_ref[...], preferred_element_type=jnp.float32)
  ft_ref[...] = jnp.dot(xn, wt_ref[...], preferred_element_type=jnp.float32)
  fe = jnp.dot(xn, we_ref[...], preferred_element_type=jnp.float32)
  for h in range(H):
    f0_ref[h] = fe[:, h * DH:(h + 1) * DH]


def _k_pre(feat, wh, wt, we, g, b):
  bn = 400
  return pl.pallas_call(
      _pre_body,
      grid=(N // bn,),
      in_specs=[pl.BlockSpec((bn, D), lambda i: (i, 0)),
                pl.BlockSpec((D, D), lambda i: (0, 0)),
                pl.BlockSpec((D, D), lambda i: (0, 0)),
                pl.BlockSpec((D, D), lambda i: (0, 0)),
                pl.BlockSpec((D,), lambda i: (0,)),
                pl.BlockSpec((D,), lambda i: (0,))],
      out_specs=[pl.BlockSpec((bn, D), lambda i: (i, 0)),
                 pl.BlockSpec((bn, D), lambda i: (i, 0)),
                 pl.BlockSpec((H, bn, DH), lambda i: (0, i, 0))],
      out_shape=[jax.ShapeDtypeStruct((N, D), jnp.float32),
                 jax.ShapeDtypeStruct((N, D), jnp.float32),
                 jax.ShapeDtypeStruct((H, N, DH), jnp.float32)],
  )(feat, wh, wt, we, g, b)


@functools.partial(
    pl.kernel, mesh=_MESH, compiler_params=_SC_PARAMS,
    out_type=(jax.ShapeDtypeStruct((NT, NT, CAPB), jnp.int32),
              jax.ShapeDtypeStruct((NT, NT), jnp.int32)),
    scratch_types=[pltpu.VMEM((ESH,), jnp.int32),
                   pltpu.VMEM((NT, CAPB), jnp.int32),
                   pltpu.VMEM((48,), jnp.int32)],
)
def _k_bin(dst_hbm, bins_o, cnt_o, shard_v, bins_v, cnt_v):
  tid = lax.axis_index("c") * 16 + lax.axis_index("s")
  pltpu.sync_copy(dst_hbm.at[pl.ds(tid * ESH, ESH)], shard_v)
  for k in range(2):
    cnt_v[pl.ds(16 * k, 16)] = jnp.zeros((16,), jnp.int32)
  iota = _i16()

  def body(j, _):
    d = shard_v[pl.ds(j * 16, 16)]
    r = d // RNG
    dl = d - r * RNG
    eid = tid * ESH + j * 16 + iota
    word = eid * 512 + dl
    sk, sv = plsc.sort_key_val(r, word)
    bnd = jnp.logical_or(iota == 0, _gath16(sk, jnp.maximum(iota - 1, 0)) != sk)
    first = plsc.cummax(jnp.where(bnd, iota, -1))
    rank = iota - first
    base = plsc.load_gather(cnt_v, [sk])
    pos = jnp.minimum(base + rank, CAPB - 1)
    plsc.store_scatter(bins_v, [sk, pos], sv)
    lastm = jnp.logical_or(iota == 15, _gath16(sk, jnp.minimum(iota + 1, 15)) != sk)
    plsc.store_scatter(cnt_v, [sk], jnp.minimum(base + rank + 1, CAPB), mask=lastm)
    return 0

  lax.fori_loop(0, ESH // 16, body, 0)
  pltpu.sync_copy(bins_v, bins_o.at[tid])
  pltpu.sync_copy(cnt_v.at[pl.ds(0, NT)], cnt_o.at[tid])


@functools.partial(
    pl.kernel, mesh=_MESH, compiler_params=_SC_PARAMS,
    out_type=(jax.ShapeDtypeStruct((E, D), jnp.float32),
              jax.ShapeDtypeStruct((E, D), jnp.float32)),
    scratch_types=[pltpu.VMEM((ESH,), jnp.int32),
                   pltpu.VMEM((ESH,), jnp.int32),
                   pltpu.VMEM((2, 128, D), jnp.float32),
                   pltpu.VMEM((2, 128, D), jnp.float32),
                   pltpu.SemaphoreType.DMA,
                   pltpu.SemaphoreType.DMA,
                   pltpu.SemaphoreType.DMA,
                   pltpu.SemaphoreType.DMA],
)
def _k_gath(fh_hbm, ft_hbm, src_hbm, dst_hbm, fhs_o, fts_o,
            src_v, dst_v, hbuf, tbuf, sgh, swh, sgt, swt):
  tid = lax.axis_index("c") * 16 + lax.axis_index("s")
  base = tid * ESH
  pltpu.sync_copy(src_hbm.at[pl.ds(base, ESH)], src_v)
  pltpu.sync_copy(dst_hbm.at[pl.ds(base, ESH)], dst_v)

  nb_full = ESH // 128
  sizes = [128] * nb_full + [16]
  gh = {}
  wh = {}
  gt = {}
  wt = {}
  for b in range(len(sizes) + 1):
    if b < len(sizes):
      if b >= 2:
        wh[b - 2].wait()
        wt[b - 2].wait()
      off = b * 128
      nb = sizes[b]
      gh[b] = pltpu.async_copy(fh_hbm.at[src_v.at[pl.ds(off, nb)]],
                               hbuf.at[b % 2, pl.ds(0, nb)], sgh)
      gt[b] = pltpu.async_copy(ft_hbm.at[dst_v.at[pl.ds(off, nb)]],
                               tbuf.at[b % 2, pl.ds(0, nb)], sgt)
    if b >= 1:
      p = b - 1
      off = p * 128
      nb = sizes[p]
      gh[p].wait()
      wh[p] = pltpu.async_copy(hbuf.at[p % 2, pl.ds(0, nb)],
                               fhs_o.at[pl.ds(base + off, nb)], swh)
      gt[p].wait()
      wt[p] = pltpu.async_copy(tbuf.at[p % 2, pl.ds(0, nb)],
                               fts_o.at[pl.ds(base + off, nb)], swt)
  wh[len(sizes) - 1].wait()
  wt[len(sizes) - 1].wait()
  wh[len(sizes) - 2].wait()
  wt[len(sizes) - 2].wait()


def _escore_body(fhs_ref, fts_ref, amat_ref, e_ref):
  s = fhs_ref[...] + fts_ref[...]
  l = jnp.maximum(s, SLOPE * s)
  res = jnp.dot(l, amat_ref[...], preferred_element_type=jnp.float32)
  e_ref[...] = res.T


def _k_escore(fhs, fts, amat):
  be = 2560
  return pl.pallas_call(
      _escore_body,
      grid=(E // be,),
      in_specs=[pl.BlockSpec((be, D), lambda i: (i, 0)),
                pl.BlockSpec((be, D), lambda i: (i, 0)),
                pl.BlockSpec((D, H), lambda i: (0, 0))],
      out_specs=pl.BlockSpec((H, be), lambda i: (0, i)),
      out_shape=jax.ShapeDtypeStruct((H, E), jnp.float32),
  )(fhs, fts, amat)


@functools.partial(
    pl.kernel, mesh=_MESH, compiler_params=_SC_PARAMS,
    out_type=(jax.ShapeDtypeStruct((H, NT, CAPS), jnp.int32),
              jax.ShapeDtypeStruct((H, NT, CAPS), jnp.int32),
              jax.ShapeDtypeStruct((H, NT, CAPS), jnp.float32),
              jax.ShapeDtypeStruct((NT, H), jnp.int32)),
    scratch_types=[pltpu.VMEM((1040,), jnp.int32),
                   pltpu.VMEM((CAPB,), jnp.int32),
                   pltpu.VMEM((CAPM + 16,), jnp.int32),
                   pltpu.VMEM((CAPM + 16,), jnp.int32),
                   pltpu.VMEM((CAPM + 16,), jnp.int32),
                   pltpu.VMEM((CAPM + 16,), jnp.int32),
                   pltpu.VMEM((CAPM + 16,), jnp.float32),
                   pltpu.VMEM((6, 320), jnp.float32),
                   pltpu.VMEM((320,), jnp.float32),
                   pltpu.VMEM((320,), jnp.float32),
                   pltpu.VMEM((320,), jnp.float32),
                   pltpu.VMEM((CAPS + 16,), jnp.int32),
                   pltpu.VMEM((CAPS + 16,), jnp.int32),
                   pltpu.VMEM((CAPS + 16,), jnp.float32),
                   pltpu.VMEM((16,), jnp.int32),
                   pltpu.SemaphoreType.DMA],
)
def _k_seg(e_hbm, src_hbm, bins_hbm, cnt_hbm,
           ssrc_o, sdl_o, sw_o, scnt_o,
           cnt_v, bin_v, eid_v, dl_v, src_v, idx_v, ecol_v,
           tabs_v, deg_v, ctab_v, den_v, stsrc_v, stdl_v, stw_v, c8_v, sem):
  rid = lax.axis_index("c") * 16 + lax.axis_index("s")
  iota = _i16()
  zf16 = jnp.zeros((16,), jnp.float32)
  neg16 = jnp.full((16,), NEG, jnp.float32)

  for t in range(NT):
    pltpu.sync_copy(cnt_hbm.at[t], cnt_v.at[pl.ds(t * NT, NT)])

  def zbody(j, _):
    eid_v[pl.ds(j * 16, 16)] = jnp.zeros((16,), jnp.int32)
    dl_v[pl.ds(j * 16, 16)] = jnp.zeros((16,), jnp.int32)
    idx_v[pl.ds(j * 16, 16)] = jnp.zeros((16,), jnp.int32)
    return 0
  lax.fori_loop(0, (CAPM + 16) // 16, zbody, 0)

  def compact_t(t, m):
    nt = cnt_v[pl.ds(t * 32 + rid, 16)][0]
    pltpu.sync_copy(bins_hbm.at[t, rid], bin_v)

    def cbody(k, m):
      w = bin_v[pl.ds(k * 16, 16)]
      valid = k * 16 + iota < nt
      plsc.store_compressed(eid_v.at[pl.ds(m, 16)], w, mask=valid)
      return m + plsc.all_reduce_population_count(valid)[0]

    return lax.fori_loop(0, (nt + 15) // 16, cbody, m)

  m_tot = 0
  for t in range(NT):
    m_tot = compact_t(t, m_tot)
  m_tot = jnp.minimum(m_tot, CAPM)
  nch = (m_tot + 15) // 16

  for k in range(20):
    deg_v[pl.ds(k * 16, 16)] = zf16

  def degbody(j, _):
    dl = jax.lax.bitwise_and(eid_v[pl.ds(j * 16, 16)], 511)
    valid = j * 16 + iota < m_tot
    _seg_rmw(deg_v, jnp.where(valid, dl, 0),
             jnp.where(valid, 1.0, 0.0), "add")
    return 0
  lax.fori_loop(0, nch, degbody, 0)

  def pbody(j, _):
    dl = jax.lax.bitwise_and(eid_v[pl.ds(j * 16, 16)], 511)
    sk, perm = plsc.sort_key_val(dl, iota)
    pk = jax.lax.shift_left(sk, 4) + perm
    for bi, s in enumerate((1, 2, 4, 8)):
      same = jnp.logical_and(_gath16(sk, jnp.maximum(iota - s, 0)) == sk,
                             iota >= s)
      pk = pk + jax.lax.shift_left(same.astype(jnp.int32), 13 + bi)
    lastm = jnp.logical_or(iota == 15,
                           sk != _gath16(sk, jnp.minimum(iota + 1, 15)))
    pk = pk + jax.lax.shift_left(lastm.astype(jnp.int32), 17)
    dl_v[pl.ds(j * 16, 16)] = pk
    return 0
  lax.fori_loop(0, nch, pbody, 0)

  def logbody(k, _):
    dg = jnp.maximum(deg_v[pl.ds(k * 16, 16)], 1.0)
    bits = plsc.bitcast(dg, jnp.int32)
    ex = jax.lax.shift_right_logical(bits, 23) - 127
    mant = plsc.bitcast(jax.lax.bitwise_or(
        jax.lax.bitwise_and(bits, 0x007FFFFF), 0x3F800000), jnp.float32) - 1.0
    y = (ex.astype(jnp.float32) + mant) * 0.6931472
    for _ in range(3):
      y = y + dg * jnp.exp(-y) - 1.0
    ctab_v[pl.ds(k * 16, 16)] = y * (1.0 / DH)
    return 0
  lax.fori_loop(0, 20, logbody, 0)

  nb_m = (m_tot + 127) // 128

  def uib(j, _):
    idx_v[pl.ds(j * 16, 16)] = jax.lax.shift_right_logical(
        eid_v[pl.ds(j * 16, 16)], 9)
    return 0
  lax.fori_loop(0, nch, uib, 0)

  def srcb(bi, _):
    pltpu.async_copy(src_hbm.at[idx_v.at[pl.ds(bi * 128, 128)]],
                     src_v.at[pl.ds(bi * 128, 128)], sem)
    return 0
  lax.fori_loop(0, nb_m, srcb, 0)

  def srcd(bi, _):
    pltpu.make_async_copy(src_hbm.at[pl.ds(0, 128)],
                          src_v.at[pl.ds(0, 128)], sem).wait()
    return 0
  lax.fori_loop(0, nb_m, srcd, 0)

  c8_v[pl.ds(0, 16)] = jnp.zeros((16,), jnp.int32)

  def head_body(h, _):
    def ib(j, _):
      idx_v[pl.ds(j * 16, 16)] = jax.lax.shift_right_logical(
          eid_v[pl.ds(j * 16, 16)], 9) + h * E
      return 0
    lax.fori_loop(0, nch, ib, 0)

    def eb(bi, _):
      pltpu.async_copy(e_hbm.at[idx_v.at[pl.ds(bi * 128, 128)]],
                       ecol_v.at[pl.ds(bi * 128, 128)], sem)
      return 0
    lax.fori_loop(0, nb_m, eb, 0)

    def ebd(bi, _):
      pltpu.make_async_copy(e_hbm.at[pl.ds(0, 128)],
                            ecol_v.at[pl.ds(0, 128)], sem).wait()
      return 0
    lax.fori_loop(0, nb_m, ebd, 0)
    ecol_v[pl.ds(m_tot, 16)] = neg16

    for r in range(TOPK):
      def tinit(k, _, r=r):
        tabs_v[r, pl.ds(k * 16, 16)] = neg16
        return 0
      lax.fori_loop(0, 20, tinit, 0)

      def rbody(j, _, r=r):
        pk = dl_v[pl.ds(j * 16, 16)]
        b = ecol_v[pl.ds(j * 16, 16)]
        bp = _gath16(b, jax.lax.bitwise_and(pk, 15))
        if r == 0:
          val = bp
        else:
          sk = jax.lax.bitwise_and(jax.lax.shift_right_logical(pk, 4), 511)
          prev = plsc.load_gather(tabs_v.at[r - 1], [sk])
          val = jnp.where(bp < prev, bp, NEG)
        _seg_rmw_pre(tabs_v.at[r], pk, val, "max")
        return 0
      lax.fori_loop(0, nch, rbody, 0)

    def dinit(k, _):
      den_v[pl.ds(k * 16, 16)] = zf16
      return 0
    lax.fori_loop(0, 20, dinit, 0)

    def dbody(j, _):
      pk = dl_v[pl.ds(j * 16, 16)]
      b = ecol_v[pl.ds(j * 16, 16)]
      bp = _gath16(b, jax.lax.bitwise_and(pk, 15))
      sk = jax.lax.bitwise_and(jax.lax.shift_right_logical(pk, 4), 511)
      thr = plsc.load_gather(tabs_v.at[TOPK - 1], [sk])
      mx = plsc.load_gather(tabs_v.at[0], [sk])
      cc = plsc.load_gather(ctab_v, [sk])
      sel = jnp.logical_and(bp >= thr, bp > -1.0e38)
      v = jnp.where(sel, jnp.exp(cc * (bp - mx)), 0.0)
      _seg_rmw_pre(den_v, pk, v, "add")
      return 0
    lax.fori_loop(0, nch, dbody, 0)

    def ebody(j, cnt):
      pk = dl_v[pl.ds(j * 16, 16)]
      b = ecol_v[pl.ds(j * 16, 16)]
      perm = jax.lax.bitwise_and(pk, 15)
      bp = _gath16(b, perm)
      sk = jax.lax.bitwise_and(jax.lax.shift_right_logical(pk, 4), 511)
      thr = plsc.load_gather(tabs_v.at[TOPK - 1], [sk])
      mx = plsc.load_gather(tabs_v.at[0], [sk])
      cc = plsc.load_gather(ctab_v, [sk])
      dn = plsc.load_gather(den_v, [sk])
      sel = jnp.logical_and(bp >= thr, bp > -1.0e38)
      w = jnp.exp(cc * (bp - mx)) / jnp.maximum(dn, 1e-38)
      sv = _gath16(src_v[pl.ds(j * 16, 16)], perm)
      cnt = jnp.minimum(cnt, CAPS)
      plsc.store_compressed(stsrc_v.at[pl.ds(cnt, 16)], sv, mask=sel)
      plsc.store_compressed(stdl_v.at[pl.ds(cnt, 16)], sk, mask=sel)
      plsc.store_compressed(stw_v.at[pl.ds(cnt, 16)], w, mask=sel)
      return cnt + plsc.all_reduce_population_count(sel)[0]
    cnt = lax.fori_loop(0, nch, ebody, 0)
    cnt = jnp.minimum(cnt, CAPS)

    pltpu.sync_copy(stsrc_v.at[pl.ds(0, CAPS)], ssrc_o.at[h, rid])
    pltpu.sync_copy(stdl_v.at[pl.ds(0, CAPS)], sdl_o.at[h, rid])
    pltpu.sync_copy(stw_v.at[pl.ds(0, CAPS)], sw_o.at[h, rid])
    c8_v[pl.ds(0, 16)] = jnp.where(iota == h, cnt, c8_v[pl.ds(0, 16)])
    return 0

  lax.fori_loop(0, H, head_body, 0)
  pltpu.sync_copy(c8_v.at[pl.ds(0, H)], scnt_o.at[rid])


def _hop_body(f_hbm, f0_hbm, ssrc_hbm, sdl_hbm, sw_hbm, scnt_hbm, out_o,
              agg_v, idx_v, gbuf, srcb_v, dlb_v, wb_v, c8_v,
              ab_v, fb_v, sem, last):
  rid = lax.axis_index("c") * 16 + lax.axis_index("s")
  lo = rid * RNG
  iota = _i16()

  z16 = jnp.zeros((16,), jnp.float32)

  def zb(j, _):
    for u in range(8):
      agg_v[pl.ds(j * 128 + u * 16, 16)] = z16
    return 0
  lax.fori_loop(0, (H * RNG * DH) // 128, zb, 0)
  def zt(j, _):
    agg_v[pl.ds((H * RNG * DH) // 128 * 128 + j * 16, 16)] = z16
    return 0
  lax.fori_loop(0, (H * RNG * DH) % 128 // 16, zt, 0)

  pltpu.sync_copy(scnt_hbm.at[rid], c8_v.at[pl.ds(0, H)])
  call = c8_v[...]

  for h in range(H):
    nsel = call[h]
    nbb = (nsel + 127) // 128
    d1 = pltpu.async_copy(ssrc_hbm.at[h, rid], srcb_v, sem)
    d2 = pltpu.async_copy(sdl_hbm.at[h, rid], dlb_v, sem)
    d3 = pltpu.async_copy(sw_hbm.at[h, rid], wb_v, sem)
    d1.wait(); d2.wait(); d3.wait()

    def ibody(sub, _, h=h):
      o = sub * 16
      valid = o + iota < nsel
      sv = jnp.where(valid, srcb_v[pl.ds(o, 16)], 0)
      idx_v[pl.ds(o, 16)] = sv + h * N
      return 0
    lax.fori_loop(0, nbb * 8, ibody, 0)

    def gfire(bi, _):
      pltpu.async_copy(f_hbm.at[idx_v.at[pl.ds(bi * 128, 128)]],
                       gbuf.at[pl.ds(bi * 128, 128)], sem)
      return 0
    lax.fori_loop(0, nbb, gfire, 0)

    def gdrain(bi, _):
      pltpu.make_async_copy(f_hbm.at[pl.ds(0, 128)],
                            gbuf.at[pl.ds(0, 128)], sem).wait()
      return 0
    lax.fori_loop(0, nbb, gdrain, 0)

    def ssub(sub, _, h=h):
      o = sub * 16
      valid = o + iota < nsel
      wv = jnp.where(valid, wb_v[pl.ds(o, 16)], 0.0)
      dv = jnp.where(valid, dlb_v[pl.ds(o, 16)], 0)
      base16 = dv * DH + h * (RNG * DH)
      for j in range(16):
        row = gbuf[o + j, :] * wv[j]
        b = base16[j]
        agg_v[pl.ds(b, 16)] = agg_v[pl.ds(b, 16)] + row
      return 0
    lax.fori_loop(0, nbb * 8, ssub, 0)

  for h in range(H):
    pltpu.sync_copy(f0_hbm.at[pl.ds(h * N + lo, RNG)], fb_v.at[pl.ds(0, RNG)])

    def blend(j, _, h=h):
      a = agg_v[pl.ds(h * (RNG * DH) + j * DH, 16)]
      ab_v[j, :] = (1.0 - ALPHA) * a + ALPHA * fb_v[j, :]
      return 0
    lax.fori_loop(0, RNG, blend, 0)
    if last:
      pltpu.sync_copy(ab_v.at[pl.ds(0, RNG)],
                      out_o.at[pl.ds(lo, RNG), pl.ds(h * DH, DH)])
    else:
      pltpu.sync_copy(ab_v.at[pl.ds(0, RNG)],
                      out_o.at[pl.ds(h * N + lo, RNG)])


def _make_hop(last):
  out_ty = (jax.ShapeDtypeStruct((N, D), jnp.float32) if last
            else jax.ShapeDtypeStruct((H * N, DH), jnp.float32))
  return functools.partial(
      pl.kernel, mesh=_MESH, compiler_params=_SC_PARAMS,
      out_type=out_ty,
      scratch_types=[pltpu.VMEM((H * RNG * DH,), jnp.float32),
                     pltpu.VMEM((CAPS,), jnp.int32),
                     pltpu.VMEM((CAPS, DH), jnp.float32),
                     pltpu.VMEM((CAPS,), jnp.int32),
                     pltpu.VMEM((CAPS,), jnp.int32),
                     pltpu.VMEM((CAPS,), jnp.float32),
                     pltpu.VMEM((16,), jnp.int32),
                     pltpu.VMEM((320, DH), jnp.float32),
                     pltpu.VMEM((320, DH), jnp.float32),
                     pltpu.SemaphoreType.DMA],
  )(functools.partial(_hop_body, last=last))


_k_hop_mid = _make_hop(False)
_k_hop_last = _make_hop(True)


def _ffn_body(f_ref, feat_ref, g_ref, b_ref, w1_ref, b1_ref, w2_ref, b2_ref,
              out_ref):
  rst = f_ref[...] + feat_ref[...]
  mu = jnp.mean(rst, axis=-1, keepdims=True)
  var = jnp.mean(jnp.square(rst - mu), axis=-1, keepdims=True)
  y = (rst - mu) * jax.lax.rsqrt(var + 1e-5) * g_ref[...] + b_ref[...]
  hdn = jnp.maximum(
      jnp.dot(y, w1_ref[...], preferred_element_type=jnp.float32)
      + b1_ref[...], 0.0)
  out_ref[...] = (jnp.dot(hdn, w2_ref[...], preferred_element_type=jnp.float32)
                  + b2_ref[...] + rst)


def _k_ffn(f2d, feat, g, b, w1, b1, w2, b2):
  bn = 400
  return pl.pallas_call(
      _ffn_body,
      grid=(N // bn,),
      in_specs=[pl.BlockSpec((bn, D), lambda i: (i, 0)),
                pl.BlockSpec((bn, D), lambda i: (i, 0)),
                pl.BlockSpec((D,), lambda i: (0,)),
                pl.BlockSpec((D,), lambda i: (0,)),
                pl.BlockSpec((D, 4 * D), lambda i: (0, 0)),
                pl.BlockSpec((4 * D,), lambda i: (0,)),
                pl.BlockSpec((4 * D, D), lambda i: (0, 0)),
                pl.BlockSpec((D,), lambda i: (0,))],
      out_specs=pl.BlockSpec((bn, D), lambda i: (i, 0)),
      out_shape=jax.ShapeDtypeStruct((N, D), jnp.float32),
  )(f2d, feat, g, b, w1, b1, w2, b2)


def kernel(feat, edge_index, W_head, W_tail, W_ent, attn,
           ln1_g, ln1_b, ln2_g, ln2_b, W_ff1, b_ff1, W_ff2, b_ff2):
  src = edge_index[0].astype(jnp.int32)
  dst = edge_index[1].astype(jnp.int32)
  attn2 = attn.reshape(H, DH)
  amat = (jnp.eye(H, dtype=jnp.float32)[:, None, :]
          * attn2[:, :, None]).reshape(D, H)

  fh, ft, f0 = _k_pre(feat, W_head, W_tail, W_ent, ln1_g, ln1_b)
  f0_flat = f0.reshape(H * N, DH)

  bins, bcnt = _k_bin(dst)
  fhs, fts = _k_gath(fh, ft, src, dst)
  e = _k_escore(fhs, fts, amat).reshape(H * E)

  ssrc, sdl, sw, scnt = _k_seg(e, src, bins, bcnt)

  f = f0_flat
  for _ in range(HOP - 1):
    f = _k_hop_mid(f, f0_flat, ssrc, sdl, sw, scnt)
  f2d = _k_hop_last(f, f0_flat, ssrc, sdl, sw, scnt)

  return _k_ffn(f2d, feat, ln2_g, ln2_b, W_ff1, b_ff1, W_ff2, b_ff2)
